# baseline (speedup 1.0000x reference)
import functools

import jax
import jax.numpy as jnp
from jax.experimental import pallas as pl
from jax.experimental.pallas import tpu as pltpu

_PREC = jax.lax.Precision.HIGHEST

_N = 10000
_E = 320000
_H = 128
_G = 256
_L = 3
_NF = 8
_FH = 16
_OPH = 64

_BN = 2000
_BE = 4000


def _dot(a, b, dims=None):
    if dims is None:
        return jax.lax.dot(a, b, precision=_PREC,
                           preferred_element_type=jnp.float32)
    return jax.lax.dot_general(a, b, dims, precision=_PREC,
                               preferred_element_type=jnp.float32)


def _node_update_body(x0_ref, agg_ref, w0a_ref, w0b_ref, w1a_ref, wfil_ref,
                      h_ref, u_ref, fil_ref):
    x0 = x0_ref[...]
    t = jax.nn.relu(_dot(x0 + agg_ref[...], w0a_ref[...]))
    h = jax.nn.relu(_dot(t, w0b_ref[...]))
    h_ref[...] = h
    u_ref[...] = _dot(x0, w1a_ref[...])
    fil_ref[...] = jax.nn.sigmoid(_dot(h, wfil_ref[...]))


def _node_update(x0, agg, w0a, w0b, w1a, wfil):
    nb = _N // _BN
    return pl.pallas_call(
        _node_update_body,
        grid=(nb,),
        in_specs=[
            pl.BlockSpec((_BN, _H), lambda i: (i, 0)),
            pl.BlockSpec((_BN, _H), lambda i: (i, 0)),
            pl.BlockSpec((_H, _H), lambda i: (0, 0)),
            pl.BlockSpec((_H, _H), lambda i: (0, 0)),
            pl.BlockSpec((_H, _H), lambda i: (0, 0)),
            pl.BlockSpec((_H, _NF), lambda i: (0, 0)),
        ],
        out_specs=[
            pl.BlockSpec((_BN, _H), lambda i: (i, 0)),
            pl.BlockSpec((_BN, _H), lambda i: (i, 0)),
            pl.BlockSpec((_BN, _NF), lambda i: (i, 0)),
        ],
        out_shape=[
            jax.ShapeDtypeStruct((_N, _H), jnp.float32),
            jax.ShapeDtypeStruct((_N, _H), jnp.float32),
            jax.ShapeDtypeStruct((_N, _NF), jnp.float32),
        ],
    )(x0, agg, w0a, w0b, w1a, wfil)


def _edge_mlp_body(x1_ref, gsum_ref, w1a_ref, w1b_ref, out_ref):
    z = _dot(x1_ref[...], w1a_ref[...]) + gsum_ref[...]
    out_ref[...] = jax.nn.relu(_dot(jax.nn.relu(z), w1b_ref[...]))


def _edge_mlp(x1, gsum, w1a, w1b):
    nb = _E // _BE
    return pl.pallas_call(
        _edge_mlp_body,
        grid=(nb,),
        in_specs=[
            pl.BlockSpec((_BE, _H), lambda i: (i, 0)),
            pl.BlockSpec((_BE, _H), lambda i: (i, 0)),
            pl.BlockSpec((_H, _H), lambda i: (0, 0)),
            pl.BlockSpec((_H, _H), lambda i: (0, 0)),
        ],
        out_specs=pl.BlockSpec((_BE, _H), lambda i: (i, 0)),
        out_shape=jax.ShapeDtypeStruct((_E, _H), jnp.float32),
    )(x1, gsum, w1a, w1b)


def _edge_mlp_last_body(x1_ref, gsum_ref, w1a_ref, w1b_ref, bsrc_ref,
                        out_ref, g1_ref):
    z = _dot(x1_ref[...], w1a_ref[...]) + gsum_ref[...]
    h = jax.nn.relu(_dot(jax.nn.relu(z), w1b_ref[...]))
    out_ref[...] = h
    onehot = (bsrc_ref[...] == jax.lax.broadcasted_iota(
        jnp.int32, (_BE, _G), 1)).astype(jnp.float32)
    part = _dot(onehot, h, (((0,), (0,)), ((), ())))

    @pl.when(pl.program_id(0) == 0)
    def _():
        g1_ref[...] = jnp.zeros_like(g1_ref)

    g1_ref[...] += part


def _edge_mlp_last(x1, gsum, w1a, w1b, bsrc2d):
    nb = _E // _BE
    return pl.pallas_call(
        _edge_mlp_last_body,
        grid=(nb,),
        in_specs=[
            pl.BlockSpec((_BE, _H), lambda i: (i, 0)),
            pl.BlockSpec((_BE, _H), lambda i: (i, 0)),
            pl.BlockSpec((_H, _H), lambda i: (0, 0)),
            pl.BlockSpec((_H, _H), lambda i: (0, 0)),
            pl.BlockSpec((_BE, 1), lambda i: (i, 0)),
        ],
        out_specs=[
            pl.BlockSpec((_BE, _H), lambda i: (i, 0)),
            pl.BlockSpec((_G, _H), lambda i: (0, 0)),
        ],
        out_shape=[
            jax.ShapeDtypeStruct((_E, _H), jnp.float32),
            jax.ShapeDtypeStruct((_G, _H), jnp.float32),
        ],
    )(x1, gsum, w1a, w1b, bsrc2d)


def _pe_body(fil_ref, death_ref, r_ref, w0r_ref, w1r_ref, wp2e_ref, batch_ref,
             ph_ref):
    fil128 = _dot(fil_ref[...], r_ref[...])
    death128 = _dot(death_ref[...], r_ref[...])
    m = jax.nn.relu(fil128 * w0r_ref[...] + death128 * w1r_ref[...])
    pe = _dot(m, wp2e_ref[...])
    onehot = (batch_ref[...] == jax.lax.broadcasted_iota(
        jnp.int32, (_BN, _G), 1)).astype(jnp.float32)

    @pl.when(pl.program_id(0) == 0)
    def _():
        ph_ref[...] = jnp.zeros_like(ph_ref)

    ph_ref[...] += _dot(onehot, pe, (((0,), (0,)), ((), ())))


def _pe_last_body(fil_ref, death_ref, r_ref, w0r_ref, w1r_ref, wp2e_ref,
                  batch_ref, x0_ref, ph_ref, g0_ref):
    fil128 = _dot(fil_ref[...], r_ref[...])
    death128 = _dot(death_ref[...], r_ref[...])
    m = jax.nn.relu(fil128 * w0r_ref[...] + death128 * w1r_ref[...])
    pe = _dot(m, wp2e_ref[...])
    onehot = (batch_ref[...] == jax.lax.broadcasted_iota(
        jnp.int32, (_BN, _G), 1)).astype(jnp.float32)

    @pl.when(pl.program_id(0) == 0)
    def _():
        ph_ref[...] = jnp.zeros_like(ph_ref)
        g0_ref[...] = jnp.zeros_like(g0_ref)

    ph_ref[...] += _dot(onehot, pe, (((0,), (0,)), ((), ())))
    g0_ref[...] += _dot(onehot, x0_ref[...], (((0,), (0,)), ((), ())))


_pe_common_specs = [
    pl.BlockSpec((_BN, _NF), lambda i: (i, 0)),
    pl.BlockSpec((_BN, _NF), lambda i: (i, 0)),
    pl.BlockSpec((_NF, _H), lambda i: (0, 0)),
    pl.BlockSpec((1, _H), lambda i: (0, 0)),
    pl.BlockSpec((1, _H), lambda i: (0, 0)),
    pl.BlockSpec((_H, _OPH), lambda i: (0, 0)),
    pl.BlockSpec((_BN, 1), lambda i: (i, 0)),
]


def _pe_readout(fil, death, r, w0r, w1r, wp2e, batch2d):
    nb = _N // _BN
    return pl.pallas_call(
        _pe_body,
        grid=(nb,),
        in_specs=list(_pe_common_specs),
        out_specs=pl.BlockSpec((_G, _OPH), lambda i: (0, 0)),
        out_shape=jax.ShapeDtypeStruct((_G, _OPH), jnp.float32),
    )(fil, death, r, w0r, w1r, wp2e, batch2d)


def _pe_readout_last(fil, death, r, w0r, w1r, wp2e, batch2d, x0):
    nb = _N // _BN
    return pl.pallas_call(
        _pe_last_body,
        grid=(nb,),
        in_specs=list(_pe_common_specs) + [
            pl.BlockSpec((_BN, _H), lambda i: (i, 0)),
        ],
        out_specs=[
            pl.BlockSpec((_G, _OPH), lambda i: (0, 0)),
            pl.BlockSpec((_G, _H), lambda i: (0, 0)),
        ],
        out_shape=[
            jax.ShapeDtypeStruct((_G, _OPH), jnp.float32),
            jax.ShapeDtypeStruct((_G, _H), jnp.float32),
        ],
    )(fil, death, r, w0r, w1r, wp2e, batch2d, x0)


def _head_body(g0_ref, g1_ref, ph0_ref, ph1_ref, ph2_ref, wl0_ref, b0_ref,
               wl1_ref, b1_ref, w2a_ref, w2b_ref, b2_ref, out_ref):
    z = (jax.nn.relu(_dot(g0_ref[...], wl0_ref[...]) + b0_ref[...]) +
         jax.nn.relu(_dot(g1_ref[...], wl1_ref[...]) + b1_ref[...]))
    ph = (ph0_ref[...] + ph1_ref[...] + ph2_ref[...]) * (1.0 / 3.0)
    out_ref[...] = _dot(z, w2a_ref[...]) + _dot(ph, w2b_ref[...]) + b2_ref[...]


def _head(g0, g1, ph0, ph1, ph2, wl0, b0, wl1, b1, w2a, w2b, b2):
    return pl.pallas_call(
        _head_body,
        out_shape=jax.ShapeDtypeStruct((_G, 1), jnp.float32),
    )(g0, g1, ph0, ph1, ph2, wl0, b0, wl1, b1, w2a, w2b, b2)


def kernel(atom_ids, edge_index, batch, atom_table, W0a, W0b, W1a, W1b, Wfil,
           Wp1, Wp2, Wlin1_0, b1_0, Wlin1_1, b1_1, Wlin2, blin2):
    src = edge_index[0]
    dst = edge_index[1]

    x0 = jnp.take(atom_table, atom_ids, axis=0)
    x1 = x0[src] + x0[dst]
    bsrc2d = batch[src].astype(jnp.int32).reshape(_E, 1)
    batch2d = batch.astype(jnp.int32).reshape(_N, 1)

    eye8 = jnp.eye(_NF, dtype=jnp.float32)
    r = jnp.repeat(eye8, _FH, axis=1)
    w0r = jnp.tile(Wp1[:, 0, :], (1, _NF)).reshape(_L, 1, _NF * _FH)
    w1r = jnp.tile(Wp1[:, 1, :], (1, _NF)).reshape(_L, 1, _NF * _FH)
    wp2e = jnp.tile(Wp2, (1, _NF, 1)) / float(_NF)

    ph_list = []
    g0 = g1 = None
    for l in range(_L):
        agg = jnp.zeros_like(x0).at[dst].add(x0[src])
        h, u, fil = _node_update(x0, agg, W0a[l], W0b[l], W1a[l], Wfil[l])
        gsum = u[src] + u[dst]
        if l == _L - 1:
            x1, g1 = _edge_mlp_last(x1, gsum, W1a[l], W1b[l], bsrc2d)
        else:
            x1 = _edge_mlp(x1, gsum, W1a[l], W1b[l])
        x0 = h
        efil = jnp.maximum(fil[src], fil[dst])
        death = jnp.zeros((_N, _NF), jnp.float32).at[dst].max(efil)
        if l == _L - 1:
            ph_l, g0 = _pe_readout_last(fil, death, r, w0r[l], w1r[l],
                                        wp2e[l], batch2d, x0)
        else:
            ph_l = _pe_readout(fil, death, r, w0r[l], w1r[l], wp2e[l], batch2d)
        ph_list.append(ph_l)

    b0_2d = b1_0.reshape(1, 2 * _H)
    b1_2d = b1_1.reshape(1, 2 * _H)
    w2a = Wlin2[:2 * _H]
    w2b = Wlin2[2 * _H:]
    b2_2d = blin2.reshape(1, 1)
    return _head(g0, g1, ph_list[0], ph_list[1], ph_list[2],
                 Wlin1_0, b0_2d, Wlin1_1, b1_2d, w2a, w2b, b2_2d)


# speedup vs baseline: 1.9884x; 1.9884x over previous
import functools

import jax
import jax.numpy as jnp
from jax import lax
from jax.experimental import pallas as pl
from jax.experimental.pallas import tpu as pltpu
from jax.experimental.pallas import tpu_sc as plsc

_PREC = jax.lax.Precision.HIGHEST

_N = 10000
_E = 320000
_H = 128
_G = 256
_L = 3
_NF = 8
_FH = 16
_OPH = 64

_BN = 2000

_NW = 32
_EP = 327680
_CW = _EP // _NW
_C = 128
_NCH = _CW // _C

_BE = 4096


def _sc_mesh():
    return plsc.VectorSubcoreMesh(core_axis_name="c", subcore_axis_name="s")


def _wid():
    return lax.axis_index("s") * 2 + lax.axis_index("c")


def _gather_sum_body(table, src3, dst3, out, idx_s, idx_d, ra, rb, ro,
                     gs0, gs1, ws0, ws1):
    w = _wid()
    pltpu.sync_copy(src3.at[w], idx_s)
    pltpu.sync_copy(dst3.at[w], idx_d)
    gsems = (gs0, gs1)
    wsems = (ws0, ws1)

    def fetch(i, b):
        pltpu.make_async_copy(table.at[idx_s.at[i]], ra.at[b], gsems[b]).start()
        pltpu.make_async_copy(table.at[idx_d.at[i]], rb.at[b], gsems[b]).start()

    fetch(0, 0)
    fetch(1, 1)

    def outer(it, carry):
        i0 = it * 2
        for b in range(2):
            i = i0 + b
            pltpu.make_async_copy(table.at[idx_s.at[i]], ra.at[b],
                                  gsems[b]).wait()
            pltpu.make_async_copy(table.at[idx_d.at[i]], rb.at[b],
                                  gsems[b]).wait()

            @pl.when(i >= 2)
            def _():
                pltpu.make_async_copy(
                    ro.at[b], out.at[pl.ds(0, _C)], wsems[b]).wait()

            rab, rbb, rob = ra.at[b], rb.at[b], ro.at[b]

            def add_row(r, c2):
                for j in range(8):
                    sl = pl.ds(j * 16, 16)
                    rob[r, sl] = rab[r, sl] + rbb[r, sl]
                return c2

            lax.fori_loop(0, _C, add_row, 0, unroll=2)
            base = _wid() * _CW + i * _C
            pltpu.make_async_copy(ro.at[b], out.at[pl.ds(base, _C)],
                                  wsems[b]).start()

            @pl.when(i + 2 < _NCH)
            def _():
                fetch(i + 2, b)
        return carry

    lax.fori_loop(0, _NCH // 2, outer, 0)
    for b in range(2):
        pltpu.make_async_copy(ro.at[b], out.at[pl.ds(0, _C)], wsems[b]).wait()


_NPAD = 10240
_RPT = _NPAD // 16
_FLAT8 = _NPAD * _NF
_RFT = _FLAT8 // 16


def _scatter_add_body(table, src3, dst3, out, idx_s, idx_d, rows, acc,
                      gs0, gs1):
    w = _wid()
    c = lax.axis_index("c")
    t = lax.axis_index("s")

    def zrow(r, c2):
        for j in range(8):
            rows[0, r, pl.ds(j * 16, 16)] = jnp.zeros((16,), jnp.float32)
        return c2

    lax.fori_loop(0, _C, zrow, 0)
    for k in range(_RPT // _C):
        pltpu.sync_copy(rows.at[0], acc.at[pl.ds(t * _RPT + k * _C, _C)])
    plsc.subcore_barrier()

    gsems = (gs0, gs1)
    hch = _NCH // 2

    for half in range(2):
        pltpu.sync_copy(src3.at[w, pl.ds(half * hch, hch)], idx_s)
        pltpu.sync_copy(dst3.at[w, pl.ds(half * hch, hch)], idx_d)

        def fetch(i, b):
            pltpu.make_async_copy(table.at[idx_s.at[i]], rows.at[b],
                                  gsems[b]).start()

        fetch(0, 0)
        fetch(1, 1)
        base0 = w * _CW + half * hch * _C

        def outer(it, carry):
            i0 = it * 2
            for b in range(2):
                i = i0 + b
                pltpu.make_async_copy(table.at[idx_s.at[i]], rows.at[b],
                                      gsems[b]).wait()
                pltpu.sync_copy(rows.at[b], acc.at[idx_d.at[i]], add=True)

                @pl.when(i + 2 < hch)
                def _():
                    fetch(i + 2, b)
            return carry

        lax.fori_loop(0, hch // 2, outer, 0)

    plsc.subcore_barrier()
    pltpu.sync_copy(acc.at[pl.ds(t * _RPT, _RPT)],
                    out.at[c, pl.ds(t * _RPT, _RPT)])


def _sc_scatter_add(table, src3, dsts3):
    return pl.kernel(
        _scatter_add_body,
        mesh=_sc_mesh(),
        out_type=jax.ShapeDtypeStruct((2, _NPAD, _H), jnp.float32),
        scratch_types=[
            pltpu.VMEM((_NCH // 2, _C), jnp.int32),
            pltpu.VMEM((_NCH // 2, _C), jnp.int32),
            pltpu.VMEM((2, _C, _H), jnp.float32),
            pltpu.VMEM_SHARED((_NPAD, _H), jnp.float32),
            pltpu.SemaphoreType.DMA,
            pltpu.SemaphoreType.DMA,
        ],
    )(table, src3, dsts3)


def _death_body(fil128, src3, dx8, out, local, idx_s, dx0, dx1, rs0, rs1,
                gs0, gs1):
    w = _wid()
    pltpu.sync_copy(src3.at[w], idx_s)

    def zslot(k, c2):
        local[pl.ds(k * 16, 16)] = jnp.zeros((16,), jnp.float32)
        return c2

    lax.fori_loop(0, _FLAT8 // 16, zslot, 0)

    iota = jax.lax.iota(jnp.int32, 16)
    mlo = iota < 8
    mhi = iota >= 8
    gsems = (gs0, gs1)
    dxs = (dx0, dx1)
    rss = (rs0, rs1)

    def fetch(i, b):
        pltpu.make_async_copy(fil128.at[idx_s.at[i]], rss[b],
                              gsems[b]).start()
        pltpu.make_async_copy(dx8.at[w, i], dxs[b], gsems[b]).start()

    fetch(0, 0)
    fetch(1, 1)

    def outer(it, carry):
        i0 = it * 2
        for b in range(2):
            i = i0 + b
            pltpu.make_async_copy(fil128.at[idx_s.at[i]], rss[b],
                                  gsems[b]).wait()
            pltpu.make_async_copy(dx8.at[w, i], dxs[b], gsems[b]).wait()
            rsb, dxb = rss[b], dxs[b]

            def pair(r, c2):
                r2 = r * 2
                e0 = rsb[r2, pl.ds(0, 16)]
                e1 = rsb[r2 + 1, pl.ds(0, 16)]
                idx = dxb[pl.ds(r2 * 8, 16)]
                old = plsc.load_gather(local, [idx])
                plsc.store_scatter(local, [idx], jnp.maximum(old, e0),
                                   mask=mlo)
                old2 = plsc.load_gather(local, [idx])
                plsc.store_scatter(local, [idx], jnp.maximum(old2, e1),
                                   mask=mhi)
                return c2

            lax.fori_loop(0, _C // 2, pair, 0)

            @pl.when(i + 2 < _NCH)
            def _():
                fetch(i + 2, b)
        return carry

    lax.fori_loop(0, _NCH // 2, outer, 0)
    pltpu.sync_copy(local, out.at[w])


def _sc_death(fil128, src3, dx8):
    return pl.kernel(
        _death_body,
        mesh=_sc_mesh(),
        compiler_params=pltpu.CompilerParams(needs_layout_passes=False),
        out_type=jax.ShapeDtypeStruct((_NW, _FLAT8), jnp.float32),
        scratch_types=[
            pltpu.VMEM((_FLAT8,), jnp.float32),
            pltpu.VMEM((_NCH, _C), jnp.int32),
            pltpu.VMEM((_C * _NF,), jnp.int32),
            pltpu.VMEM((_C * _NF,), jnp.int32),
            pltpu.VMEM((_C, _H), jnp.float32),
            pltpu.VMEM((_C, _H), jnp.float32),
            pltpu.SemaphoreType.DMA,
            pltpu.SemaphoreType.DMA,
        ],
    )(fil128, src3, dx8)


def _touched_body(dst3, out, local, idx_d):
    w = _wid()
    pltpu.sync_copy(dst3.at[w], idx_d)

    def zslot(k, c2):
        local[pl.ds(k * 16, 16)] = jnp.zeros((16,), jnp.float32)
        return c2

    lax.fori_loop(0, _NPAD // 16, zslot, 0)
    ones = jnp.ones((16,), jnp.float32)

    def chunk(i, c2):
        for j in range(_C // 16):
            idx = idx_d[i, pl.ds(j * 16, 16)]
            plsc.store_scatter(local, [idx], ones)
        return c2

    lax.fori_loop(0, _NCH, chunk, 0)
    pltpu.sync_copy(local, out.at[w])


def _sc_touched(dsts3):
    return pl.kernel(
        _touched_body,
        mesh=_sc_mesh(),
        compiler_params=pltpu.CompilerParams(needs_layout_passes=False),
        out_type=jax.ShapeDtypeStruct((_NW, _NPAD), jnp.float32),
        scratch_types=[
            pltpu.VMEM((_NPAD,), jnp.float32),
            pltpu.VMEM((_NCH, _C), jnp.int32),
        ],
    )(dsts3)


def _fmax_body(*refs):
    out_ref = refs[-1]
    d = refs[0][0]
    for k in range(1, _NW):
        d = jnp.maximum(d, refs[k][0])
    out_ref[...] = d


def _flat_max32(a32r, rrows, brows):
    nb = rrows // brows
    return pl.pallas_call(
        _fmax_body,
        grid=(nb,),
        in_specs=[pl.BlockSpec((1, brows, _H), functools.partial(
            lambda k, i: (k, i, 0), k)) for k in range(_NW)],
        out_specs=pl.BlockSpec((brows, _H), lambda i: (i, 0)),
        out_shape=jax.ShapeDtypeStruct((rrows, _H), jnp.float32),
    )(*([a32r] * _NW))


def _death_merge_body(d_ref, fil128_ref, tch_ref, out_ref):
    fil = fil128_ref[...][:, :_NF]
    out_ref[...] = jnp.maximum(d_ref[...], fil * tch_ref[...])


def _death_merge(dmax, fil128, touched):
    nb = _N // _BN
    return pl.pallas_call(
        _death_merge_body,
        grid=(nb,),
        in_specs=[
            pl.BlockSpec((_BN, _NF), lambda i: (i, 0)),
            pl.BlockSpec((_BN, _H), lambda i: (i, 0)),
            pl.BlockSpec((_BN, 1), lambda i: (i, 0)),
        ],
        out_specs=pl.BlockSpec((_BN, _NF), lambda i: (i, 0)),
        out_shape=jax.ShapeDtypeStruct((_N, _NF), jnp.float32),
    )(dmax, fil128, touched)


def _sc_gather_sum(table, src3, dst3):
    return pl.kernel(
        _gather_sum_body,
        mesh=_sc_mesh(),
        out_type=jax.ShapeDtypeStruct((_EP, _H), jnp.float32),
        scratch_types=[
            pltpu.VMEM((_NCH, _C), jnp.int32),
            pltpu.VMEM((_NCH, _C), jnp.int32),
            pltpu.VMEM((2, _C, _H), jnp.float32),
            pltpu.VMEM((2, _C, _H), jnp.float32),
            pltpu.VMEM((2, _C, _H), jnp.float32),
            pltpu.SemaphoreType.DMA,
            pltpu.SemaphoreType.DMA,
            pltpu.SemaphoreType.DMA,
            pltpu.SemaphoreType.DMA,
        ],
    )(table, src3, dst3)


def _dot(a, b, dims=None):
    if dims is None:
        return jax.lax.dot(a, b, precision=_PREC,
                           preferred_element_type=jnp.float32)
    return jax.lax.dot_general(a, b, dims, precision=_PREC,
                               preferred_element_type=jnp.float32)


def _embed_body(ids_ref, tbl_ref, out_ref):
    onehot = (ids_ref[...] == jax.lax.broadcasted_iota(
        jnp.int32, (_BN, _H), 1)).astype(jnp.float32)
    out_ref[...] = _dot(onehot, tbl_ref[...])


def _embed_tc(ids2d, tbl_pad):
    nb = _N // _BN
    return pl.pallas_call(
        _embed_body,
        grid=(nb,),
        in_specs=[
            pl.BlockSpec((_BN, 1), lambda i: (i, 0)),
            pl.BlockSpec((_H, _H), lambda i: (0, 0)),
        ],
        out_specs=pl.BlockSpec((_BN, _H), lambda i: (i, 0)),
        out_shape=jax.ShapeDtypeStruct((_N, _H), jnp.float32),
    )(ids2d, tbl_pad)


def _node_update_body(x0_ref, a0_ref, a1_ref, w0a_ref, w0b_ref, w1a_ref,
                      wfil_ref, h_ref, u_ref, fil_ref):
    x0 = x0_ref[...]
    agg = a0_ref[0] + a1_ref[0]
    t = jax.nn.relu(_dot(x0 + agg, w0a_ref[...]))
    h = jax.nn.relu(_dot(t, w0b_ref[...]))
    h_ref[...] = h
    u_ref[...] = _dot(x0, w1a_ref[...])
    fil = jax.nn.sigmoid(_dot(h, wfil_ref[...]))
    fil_ref[...] = jnp.tile(fil, (1, 16))


def _node_update(x0, agg2, w0a, w0b, w1a, wfil):
    nb = _N // _BN
    return pl.pallas_call(
        _node_update_body,
        grid=(nb,),
        in_specs=[
            pl.BlockSpec((_BN, _H), lambda i: (i, 0)),
            pl.BlockSpec((1, _BN, _H), lambda i: (0, i, 0)),
            pl.BlockSpec((1, _BN, _H), lambda i: (1, i, 0)),
            pl.BlockSpec((_H, _H), lambda i: (0, 0)),
            pl.BlockSpec((_H, _H), lambda i: (0, 0)),
            pl.BlockSpec((_H, _H), lambda i: (0, 0)),
            pl.BlockSpec((_H, _NF), lambda i: (0, 0)),
        ],
        out_specs=[
            pl.BlockSpec((_BN, _H), lambda i: (i, 0)),
            pl.BlockSpec((_BN, _H), lambda i: (i, 0)),
            pl.BlockSpec((_BN, _H), lambda i: (i, 0)),
        ],
        out_shape=[
            jax.ShapeDtypeStruct((_N, _H), jnp.float32),
            jax.ShapeDtypeStruct((_N, _H), jnp.float32),
            jax.ShapeDtypeStruct((_N, _H), jnp.float32),
        ],
    )(x0, agg2, agg2, w0a, w0b, w1a, wfil)


def _edge_mlp_body(x1_ref, gsum_ref, w1a_ref, w1b_ref, out_ref):
    z = _dot(x1_ref[...], w1a_ref[...]) + gsum_ref[...]
    out_ref[...] = jax.nn.relu(_dot(jax.nn.relu(z), w1b_ref[...]))


def _edge_mlp(x1, gsum, w1a, w1b):
    nb = _EP // _BE
    return pl.pallas_call(
        _edge_mlp_body,
        grid=(nb,),
        in_specs=[
            pl.BlockSpec((_BE, _H), lambda i: (i, 0)),
            pl.BlockSpec((_BE, _H), lambda i: (i, 0)),
            pl.BlockSpec((_H, _H), lambda i: (0, 0)),
            pl.BlockSpec((_H, _H), lambda i: (0, 0)),
        ],
        out_specs=pl.BlockSpec((_BE, _H), lambda i: (i, 0)),
        out_shape=jax.ShapeDtypeStruct((_EP, _H), jnp.float32),
    )(x1, gsum, w1a, w1b)


def _pe_common(fil128_ref, death_ref, r_ref, w0r_ref, w1r_ref, wp2e_ref,
               batch_ref):
    fil = fil128_ref[...][:, :_NF]
    death = death_ref[...]
    fil128 = _dot(fil, r_ref[...])
    death128 = _dot(death, r_ref[...])
    m = jax.nn.relu(fil128 * w0r_ref[...] + death128 * w1r_ref[...])
    pe = _dot(m, wp2e_ref[...])
    onehot = (batch_ref[...] == jax.lax.broadcasted_iota(
        jnp.int32, (_BN, _G), 1)).astype(jnp.float32)
    return pe, onehot


def _pe_body(fil16_ref, death_ref, r_ref, w0r_ref, w1r_ref, wp2e_ref,
             batch_ref, ph_ref):
    pe, onehot = _pe_common(fil16_ref, death_ref, r_ref, w0r_ref,
                            w1r_ref, wp2e_ref, batch_ref)

    @pl.when(pl.program_id(0) == 0)
    def _():
        ph_ref[...] = jnp.zeros_like(ph_ref)

    ph_ref[...] += _dot(onehot, pe, (((0,), (0,)), ((), ())))


def _pe_last_body(fil16_ref, death_ref, r_ref, w0r_ref, w1r_ref,
                  wp2e_ref, batch_ref, x0_ref, s0_ref, s1_ref,
                  ph_ref, g0_ref, g1_ref):
    pe, onehot = _pe_common(fil16_ref, death_ref, r_ref, w0r_ref,
                            w1r_ref, wp2e_ref, batch_ref)

    @pl.when(pl.program_id(0) == 0)
    def _():
        ph_ref[...] = jnp.zeros_like(ph_ref)
        g0_ref[...] = jnp.zeros_like(g0_ref)
        g1_ref[...] = jnp.zeros_like(g1_ref)

    ph_ref[...] += _dot(onehot, pe, (((0,), (0,)), ((), ())))
    g0_ref[...] += _dot(onehot, x0_ref[...], (((0,), (0,)), ((), ())))
    s1 = s0_ref[0] + s1_ref[0]
    g1_ref[...] += _dot(onehot, s1, (((0,), (0,)), ((), ())))


_pe_common_specs = [
    pl.BlockSpec((_BN, _H), lambda i: (i, 0)),
    pl.BlockSpec((_BN, _NF), lambda i: (i, 0)),
    pl.BlockSpec((_NF, _H), lambda i: (0, 0)),
    pl.BlockSpec((1, _H), lambda i: (0, 0)),
    pl.BlockSpec((1, _H), lambda i: (0, 0)),
    pl.BlockSpec((_H, _OPH), lambda i: (0, 0)),
    pl.BlockSpec((_BN, 1), lambda i: (i, 0)),
]


def _pe_readout(fil16, death, r, w0r, w1r, wp2e, batch2d):
    nb = _N // _BN
    return pl.pallas_call(
        _pe_body,
        grid=(nb,),
        in_specs=list(_pe_common_specs),
        out_specs=pl.BlockSpec((_G, _OPH), lambda i: (0, 0)),
        out_shape=jax.ShapeDtypeStruct((_G, _OPH), jnp.float32),
    )(fil16, death, r, w0r, w1r, wp2e, batch2d)


def _pe_readout_last(fil16, death, r, w0r, w1r, wp2e, batch2d, x0, s1p):
    nb = _N // _BN
    return pl.pallas_call(
        _pe_last_body,
        grid=(nb,),
        in_specs=list(_pe_common_specs) + [
            pl.BlockSpec((_BN, _H), lambda i: (i, 0)),
            pl.BlockSpec((1, _BN, _H), lambda i: (0, i, 0)),
            pl.BlockSpec((1, _BN, _H), lambda i: (1, i, 0)),
        ],
        out_specs=[
            pl.BlockSpec((_G, _OPH), lambda i: (0, 0)),
            pl.BlockSpec((_G, _H), lambda i: (0, 0)),
            pl.BlockSpec((_G, _H), lambda i: (0, 0)),
        ],
        out_shape=[
            jax.ShapeDtypeStruct((_G, _OPH), jnp.float32),
            jax.ShapeDtypeStruct((_G, _H), jnp.float32),
            jax.ShapeDtypeStruct((_G, _H), jnp.float32),
        ],
    )(fil16, death, r, w0r, w1r, wp2e, batch2d, x0, s1p, s1p)


def _head_body(g0_ref, g1_ref, ph0_ref, ph1_ref, ph2_ref, wl0_ref, b0_ref,
               wl1_ref, b1_ref, w2a_ref, w2b_ref, b2_ref, out_ref):
    z = (jax.nn.relu(_dot(g0_ref[...], wl0_ref[...]) + b0_ref[...]) +
         jax.nn.relu(_dot(g1_ref[...], wl1_ref[...]) + b1_ref[...]))
    ph = (ph0_ref[...] + ph1_ref[...] + ph2_ref[...]) * (1.0 / 3.0)
    out_ref[...] = _dot(z, w2a_ref[...]) + _dot(ph, w2b_ref[...]) + b2_ref[...]


def _head(g0, g1, ph0, ph1, ph2, wl0, b0, wl1, b1, w2a, w2b, b2):
    return pl.pallas_call(
        _head_body,
        out_shape=jax.ShapeDtypeStruct((_G, 1), jnp.float32),
    )(g0, g1, ph0, ph1, ph2, wl0, b0, wl1, b1, w2a, w2b, b2)


def kernel(atom_ids, edge_index, batch, atom_table, W0a, W0b, W1a, W1b, Wfil,
           Wp1, Wp2, Wlin1_0, b1_0, Wlin1_1, b1_1, Wlin2, blin2):
    src = edge_index[0]
    dst = edge_index[1]

    pad = _EP - _E
    zpad = jnp.zeros((pad,), jnp.int32)
    spad = jnp.full((pad,), _N, jnp.int32)
    srcz = jnp.concatenate([src.astype(jnp.int32), zpad])
    dstz = jnp.concatenate([dst.astype(jnp.int32), zpad])
    src3 = srcz.reshape(_NW, _NCH, _C)
    dst3 = dstz.reshape(_NW, _NCH, _C)
    dsts = jnp.concatenate([dst.astype(jnp.int32), spad])
    dsts3 = dsts.reshape(_NW, _NCH, _C)
    srcs3 = jnp.concatenate([src.astype(jnp.int32), spad]).reshape(
        _NW, _NCH, _C)
    eidx3 = jnp.arange(_EP, dtype=jnp.int32).reshape(_NW, _NCH, _C)
    dx8 = ((dsts[:, None] * _NF) +
           jnp.arange(_NF, dtype=jnp.int32)[None, :]).reshape(
               _NW, _NCH, _C * _NF)

    ids2d = atom_ids.astype(jnp.int32).reshape(_N, 1)
    tbl_pad = jnp.zeros((_H, _H), jnp.float32).at[:119].set(atom_table)
    x0 = _embed_tc(ids2d, tbl_pad)
    x1 = _sc_gather_sum(x0, src3, dst3)
    batch2d = batch.astype(jnp.int32).reshape(_N, 1)

    eye8 = jnp.eye(_NF, dtype=jnp.float32)
    r = jnp.repeat(eye8, _FH, axis=1)
    w0r = jnp.tile(Wp1[:, 0, :], (1, _NF)).reshape(_L, 1, _NF * _FH)
    w1r = jnp.tile(Wp1[:, 1, :], (1, _NF)).reshape(_L, 1, _NF * _FH)
    wp2e = jnp.tile(Wp2, (1, _NF, 1)) / float(_NF)

    t32 = _sc_touched(dsts3).reshape(_NW, _NPAD // _H, _H)
    touched = _flat_max32(t32, _NPAD // _H, _NPAD // _H).reshape(_NPAD, 1)

    ph_list = []
    g0 = g1 = None
    for l in range(_L):
        agg2 = _sc_scatter_add(x0, src3, dsts3)
        h, u, fil128 = _node_update(x0, agg2, W0a[l], W0b[l], W1a[l], Wfil[l])
        gsum = _sc_gather_sum(u, src3, dst3)
        x1 = _edge_mlp(x1, gsum, W1a[l], W1b[l])
        x0 = h
        death32 = _sc_death(fil128, src3, dx8).reshape(
            _NW, _FLAT8 // _H, _H)
        dmax = _flat_max32(death32, _FLAT8 // _H, _H).reshape(_NPAD, _NF)
        death = _death_merge(dmax, fil128, touched)
        if l == _L - 1:
            s1p = _sc_scatter_add(x1, eidx3, srcs3)
            ph_l, g0, g1 = _pe_readout_last(fil128, death, r, w0r[l], w1r[l],
                                            wp2e[l], batch2d, x0, s1p)
        else:
            ph_l = _pe_readout(fil128, death, r, w0r[l], w1r[l], wp2e[l],
                               batch2d)
        ph_list.append(ph_l)

    b0_2d = b1_0.reshape(1, 2 * _H)
    b1_2d = b1_1.reshape(1, 2 * _H)
    w2a = Wlin2[:2 * _H]
    w2b = Wlin2[2 * _H:]
    b2_2d = blin2.reshape(1, 1)
    return _head(g0, g1, ph_list[0], ph_list[1], ph_list[2],
                 Wlin1_0, b0_2d, Wlin1_1, b1_2d, w2a, w2b, b2_2d)


# speedup vs baseline: 3.0654x; 1.5416x over previous
import functools

import jax
import jax.numpy as jnp
from jax import lax
from jax.experimental import pallas as pl
from jax.experimental.pallas import tpu as pltpu
from jax.experimental.pallas import tpu_sc as plsc

_PREC = jax.lax.Precision.HIGHEST

_N = 10000
_E = 320000
_H = 128
_G = 256
_L = 3
_NF = 8
_FH = 16
_OPH = 64

_BN = 2000

_NW = 32
_EP = 327680
_CW = _EP // _NW
_C = 128
_NCH = _CW // _C

_BE = 4096


def _sc_mesh():
    return plsc.VectorSubcoreMesh(core_axis_name="c", subcore_axis_name="s")


def _wid():
    return lax.axis_index("s") * 2 + lax.axis_index("c")


_NPAD = 10240
_RPT = _NPAD // 16
_FLAT8 = _NPAD * _NF
_RFT = _FLAT8 // 16


_C2 = 64
_NCH2 = _CW // _C2
_HCH2 = _NCH2 // 4


def _agg_bsum_body(table, src3, dst3, agg_out, bsum_out, idx_s, idx_d,
                   ra, rb, acc, gs0, gs1, ws0, ws1):
    w = _wid()
    c = lax.axis_index("c")
    t = lax.axis_index("s")

    def zrow(r, c2):
        for j in range(8):
            ra[0, r, pl.ds(j * 16, 16)] = jnp.zeros((16,), jnp.float32)
        return c2

    lax.fori_loop(0, _C2, zrow, 0)
    for k in range(_RPT // _C2):
        pltpu.sync_copy(ra.at[0], acc.at[pl.ds(t * _RPT + k * _C2, _C2)])
    plsc.subcore_barrier()

    gsems = (gs0, gs1)
    wsems = (ws0, ws1)

    for half in range(4):
        pltpu.sync_copy(src3.at[w, pl.ds(half * _HCH2, _HCH2)], idx_s)
        pltpu.sync_copy(dst3.at[w, pl.ds(half * _HCH2, _HCH2)], idx_d)

        def fetch(i, b):
            pltpu.make_async_copy(table.at[idx_s.at[i]], ra.at[b],
                                  gsems[b]).start()
            pltpu.make_async_copy(table.at[idx_d.at[i]], rb.at[b],
                                  gsems[b]).start()

        fetch(0, 0)
        fetch(1, 1)
        base0 = w * _CW + half * _HCH2 * _C2

        def outer(it, carry):
            i0 = it * 2
            for b in range(2):
                i = i0 + b
                pltpu.make_async_copy(table.at[idx_s.at[i]], ra.at[b],
                                      gsems[b]).wait()
                pltpu.make_async_copy(table.at[idx_d.at[i]], rb.at[b],
                                      gsems[b]).wait()
                rab, rbb = ra.at[b], rb.at[b]

                def add_row(rr, c2):
                    for j in range(8):
                        sl = pl.ds(j * 16, 16)
                        rbb[rr, sl] = rab[rr, sl] + rbb[rr, sl]
                    return c2

                lax.fori_loop(0, _C2, add_row, 0, unroll=2)
                pltpu.make_async_copy(
                    rb.at[b], bsum_out.at[pl.ds(base0 + i * _C2, _C2)],
                    wsems[b]).start()
                pltpu.sync_copy(ra.at[b], acc.at[idx_d.at[i]], add=True)

                @pl.when(i + 2 < _HCH2)
                def _():
                    pltpu.make_async_copy(
                        rb.at[b], bsum_out.at[pl.ds(0, _C2)], wsems[b]).wait()
                    fetch(i + 2, b)
            return carry

        lax.fori_loop(0, _HCH2 // 2, outer, 0)
        for b in range(2):
            pltpu.make_async_copy(rb.at[b], bsum_out.at[pl.ds(0, _C2)],
                                  wsems[b]).wait()

    plsc.subcore_barrier()
    pltpu.sync_copy(acc.at[pl.ds(t * _RPT, _RPT)],
                    agg_out.at[c, pl.ds(t * _RPT, _RPT)])


def _sc_agg_bsum(table, srcs3c, dsts3c):
    return pl.kernel(
        _agg_bsum_body,
        mesh=_sc_mesh(),
        out_type=[
            jax.ShapeDtypeStruct((2, _NPAD, _H), jnp.float32),
            jax.ShapeDtypeStruct((_EP, _H), jnp.float32),
        ],
        scratch_types=[
            pltpu.VMEM((_HCH2, _C2), jnp.int32),
            pltpu.VMEM((_HCH2, _C2), jnp.int32),
            pltpu.VMEM((2, _C2, _H), jnp.float32),
            pltpu.VMEM((2, _C2, _H), jnp.float32),
            pltpu.VMEM_SHARED((_NPAD, _H), jnp.float32),
            pltpu.SemaphoreType.DMA,
            pltpu.SemaphoreType.DMA,
            pltpu.SemaphoreType.DMA,
            pltpu.SemaphoreType.DMA,
        ],
    )(table, srcs3c, dsts3c)


def _bsrc_body(batchp, src3, out, batchv, idx_s, ob0, ob1, ws0, ws1):
    w = _wid()
    pltpu.sync_copy(batchp, batchv)
    pltpu.sync_copy(src3.at[w], idx_s)
    obs = (ob0, ob1)
    wsems = (ws0, ws1)

    def run(i, b):
        for j in range(_C // 16):
            sl = pl.ds(j * 16, 16)
            sidx = idx_s[i, sl]
            obs[b][sl] = plsc.load_gather(batchv, [sidx])
        pltpu.make_async_copy(obs[b], out.at[pl.ds(w * _CW + i * _C, _C)],
                              wsems[b]).start()

    def outer(it, carry):
        i0 = it * 2
        for b in range(2):
            i = i0 + b

            @pl.when(i >= 2)
            def _():
                pltpu.make_async_copy(obs[b], out.at[pl.ds(0, _C)],
                                      wsems[b]).wait()

            run(i, b)
        return carry

    lax.fori_loop(0, _NCH // 2, outer, 0)
    for b in range(2):
        pltpu.make_async_copy(obs[b], out.at[pl.ds(0, _C)], wsems[b]).wait()


def _sc_bsrc(batchp, srcs3):
    return pl.kernel(
        _bsrc_body,
        mesh=_sc_mesh(),
        compiler_params=pltpu.CompilerParams(needs_layout_passes=False),
        out_type=jax.ShapeDtypeStruct((_EP,), jnp.int32),
        scratch_types=[
            pltpu.VMEM((_NPAD,), jnp.int32),
            pltpu.VMEM((_NCH, _C), jnp.int32),
            pltpu.VMEM((_C,), jnp.int32),
            pltpu.VMEM((_C,), jnp.int32),
            pltpu.SemaphoreType.DMA,
            pltpu.SemaphoreType.DMA,
        ],
    )(batchp, srcs3)


def _death_body(fil128, src3, dx8, out, local, idx_s, dx0, dx1, rs0, rs1,
                gs0, gs1):
    w = _wid()
    pltpu.sync_copy(src3.at[w], idx_s)

    def zslot(k, c2):
        local[pl.ds(k * 16, 16)] = jnp.zeros((16,), jnp.float32)
        return c2

    lax.fori_loop(0, _FLAT8 // 16, zslot, 0)

    iota = jax.lax.iota(jnp.int32, 16)
    mlo = iota < 8
    mhi = iota >= 8
    gsems = (gs0, gs1)
    dxs = (dx0, dx1)
    rss = (rs0, rs1)

    def fetch(i, b):
        pltpu.make_async_copy(fil128.at[idx_s.at[i]], rss[b],
                              gsems[b]).start()
        pltpu.make_async_copy(dx8.at[w, i], dxs[b], gsems[b]).start()

    fetch(0, 0)
    fetch(1, 1)

    def outer(it, carry):
        i0 = it * 2
        for b in range(2):
            i = i0 + b
            pltpu.make_async_copy(fil128.at[idx_s.at[i]], rss[b],
                                  gsems[b]).wait()
            pltpu.make_async_copy(dx8.at[w, i], dxs[b], gsems[b]).wait()
            rsb, dxb = rss[b], dxs[b]

            def pair(r, c2):
                r2 = r * 2
                e0 = rsb[r2, pl.ds(0, 16)]
                e1 = rsb[r2 + 1, pl.ds(0, 16)]
                idx = dxb[pl.ds(r2 * 8, 16)]
                old = plsc.load_gather(local, [idx])
                plsc.store_scatter(local, [idx], jnp.maximum(old, e0),
                                   mask=mlo)
                old2 = plsc.load_gather(local, [idx])
                plsc.store_scatter(local, [idx], jnp.maximum(old2, e1),
                                   mask=mhi)
                return c2

            lax.fori_loop(0, _C // 2, pair, 0)

            @pl.when(i + 2 < _NCH)
            def _():
                fetch(i + 2, b)
        return carry

    lax.fori_loop(0, _NCH // 2, outer, 0)
    pltpu.sync_copy(local, out.at[w])


def _sc_death(fil128, src3, dx8):
    return pl.kernel(
        _death_body,
        mesh=_sc_mesh(),
        compiler_params=pltpu.CompilerParams(needs_layout_passes=False),
        out_type=jax.ShapeDtypeStruct((_NW, _FLAT8), jnp.float32),
        scratch_types=[
            pltpu.VMEM((_FLAT8,), jnp.float32),
            pltpu.VMEM((_NCH, _C), jnp.int32),
            pltpu.VMEM((_C * _NF,), jnp.int32),
            pltpu.VMEM((_C * _NF,), jnp.int32),
            pltpu.VMEM((_C, _H), jnp.float32),
            pltpu.VMEM((_C, _H), jnp.float32),
            pltpu.SemaphoreType.DMA,
            pltpu.SemaphoreType.DMA,
        ],
    )(fil128, src3, dx8)


def _touched_body(dst3, out, local, idx_d):
    w = _wid()
    pltpu.sync_copy(dst3.at[w], idx_d)

    def zslot(k, c2):
        local[pl.ds(k * 16, 16)] = jnp.zeros((16,), jnp.float32)
        return c2

    lax.fori_loop(0, _NPAD // 16, zslot, 0)
    ones = jnp.ones((16,), jnp.float32)

    def chunk(i, c2):
        for j in range(_C // 16):
            idx = idx_d[i, pl.ds(j * 16, 16)]
            plsc.store_scatter(local, [idx], ones)
        return c2

    lax.fori_loop(0, _NCH, chunk, 0)
    pltpu.sync_copy(local, out.at[w])


def _sc_touched(dsts3):
    return pl.kernel(
        _touched_body,
        mesh=_sc_mesh(),
        compiler_params=pltpu.CompilerParams(needs_layout_passes=False),
        out_type=jax.ShapeDtypeStruct((_NW, _NPAD), jnp.float32),
        scratch_types=[
            pltpu.VMEM((_NPAD,), jnp.float32),
            pltpu.VMEM((_NCH, _C), jnp.int32),
        ],
    )(dsts3)


def _fmax_body(*refs):
    out_ref = refs[-1]
    d = refs[0][0]
    for k in range(1, _NW):
        d = jnp.maximum(d, refs[k][0])
    out_ref[...] = d


def _flat_max32(a32r, rrows, brows):
    nb = rrows // brows
    return pl.pallas_call(
        _fmax_body,
        grid=(nb,),
        in_specs=[pl.BlockSpec((1, brows, _H), functools.partial(
            lambda k, i: (k, i, 0), k)) for k in range(_NW)],
        out_specs=pl.BlockSpec((brows, _H), lambda i: (i, 0)),
        out_shape=jax.ShapeDtypeStruct((rrows, _H), jnp.float32),
    )(*([a32r] * _NW))


def _death_merge_body(d_ref, fil128_ref, tch_ref, out_ref):
    fil = fil128_ref[...][:, :_NF]
    out_ref[...] = jnp.maximum(d_ref[...], fil * tch_ref[...])


def _death_merge(dmax, fil128, touched):
    nb = _N // _BN
    return pl.pallas_call(
        _death_merge_body,
        grid=(nb,),
        in_specs=[
            pl.BlockSpec((_BN, _NF), lambda i: (i, 0)),
            pl.BlockSpec((_BN, _H), lambda i: (i, 0)),
            pl.BlockSpec((_BN, 1), lambda i: (i, 0)),
        ],
        out_specs=pl.BlockSpec((_BN, _NF), lambda i: (i, 0)),
        out_shape=jax.ShapeDtypeStruct((_N, _NF), jnp.float32),
    )(dmax, fil128, touched)


def _dot(a, b, dims=None, prec=None):
    prec = prec or jax.lax.Precision.DEFAULT
    if dims is None:
        return jax.lax.dot(a, b, precision=prec,
                           preferred_element_type=jnp.float32)
    return jax.lax.dot_general(a, b, dims, precision=prec,
                               preferred_element_type=jnp.float32)


def _dotx(a, b, dims=None):
    return _dot(a, b, dims, prec=jax.lax.Precision.HIGHEST)


_BN2 = 2048


def _embed_body(ids_ref, tbl_ref, out_ref):
    onehot = (ids_ref[...] == jax.lax.broadcasted_iota(
        jnp.int32, (_BN2, _H), 1)).astype(jnp.float32)
    out_ref[...] = _dotx(onehot, tbl_ref[...])


def _embed_tc(ids2d, tbl_pad):
    nb = _NPAD // _BN2
    return pl.pallas_call(
        _embed_body,
        grid=(nb,),
        in_specs=[
            pl.BlockSpec((_BN2, 1), lambda i: (i, 0)),
            pl.BlockSpec((_H, _H), lambda i: (0, 0)),
        ],
        out_specs=pl.BlockSpec((_BN2, _H), lambda i: (i, 0)),
        out_shape=jax.ShapeDtypeStruct((_NPAD, _H), jnp.float32),
    )(ids2d, tbl_pad)


def _node_update_body(x0_ref, a0_ref, a1_ref, w0a_ref, w0b_ref,
                      wfil_ref, h_ref, fil_ref):
    x0 = x0_ref[...]
    agg = a0_ref[0] + a1_ref[0]
    t = jax.nn.relu(_dot(x0 + agg, w0a_ref[...]))
    h = jax.nn.relu(_dot(t, w0b_ref[...]))
    h_ref[...] = h
    fil = jax.nn.sigmoid(_dot(h, wfil_ref[...]))
    fil_ref[...] = jnp.tile(fil, (1, 16))


def _node_update(x0, agg2, w0a, w0b, wfil):
    nb = _NPAD // _BN2
    return pl.pallas_call(
        _node_update_body,
        grid=(nb,),
        in_specs=[
            pl.BlockSpec((_BN2, _H), lambda i: (i, 0)),
            pl.BlockSpec((1, _BN2, _H), lambda i: (0, i, 0)),
            pl.BlockSpec((1, _BN2, _H), lambda i: (1, i, 0)),
            pl.BlockSpec((_H, _H), lambda i: (0, 0)),
            pl.BlockSpec((_H, _H), lambda i: (0, 0)),
            pl.BlockSpec((_H, _NF), lambda i: (0, 0)),
        ],
        out_specs=[
            pl.BlockSpec((_BN2, _H), lambda i: (i, 0)),
            pl.BlockSpec((_BN2, _H), lambda i: (i, 0)),
        ],
        out_shape=[
            jax.ShapeDtypeStruct((_NPAD, _H), jnp.float32),
            jax.ShapeDtypeStruct((_NPAD, _H), jnp.float32),
        ],
    )(x0, agg2, agg2, w0a, w0b, wfil)


def _edge_mlp_body(x1_ref, bsum_ref, w1a_ref, w1b_ref, out_ref):
    z = _dot(x1_ref[...] + bsum_ref[...], w1a_ref[...])
    out_ref[...] = jax.nn.relu(_dot(jax.nn.relu(z), w1b_ref[...]))


def _edge_mlp(x1, bsum, w1a, w1b):
    nb = _EP // _BE
    return pl.pallas_call(
        _edge_mlp_body,
        grid=(nb,),
        in_specs=[
            pl.BlockSpec((_BE, _H), lambda i: (i, 0)),
            pl.BlockSpec((_BE, _H), lambda i: (i, 0)),
            pl.BlockSpec((_H, _H), lambda i: (0, 0)),
            pl.BlockSpec((_H, _H), lambda i: (0, 0)),
        ],
        out_specs=pl.BlockSpec((_BE, _H), lambda i: (i, 0)),
        out_shape=jax.ShapeDtypeStruct((_EP, _H), jnp.float32),
    )(x1, bsum, w1a, w1b)


def _edge_mlp0_body(bsum_ref, w1a_ref, w1b_ref, out_ref):
    z = _dot(bsum_ref[...] * 2.0, w1a_ref[...])
    out_ref[...] = jax.nn.relu(_dot(jax.nn.relu(z), w1b_ref[...]))


def _edge_mlp0(bsum, w1a, w1b):
    nb = _EP // _BE
    return pl.pallas_call(
        _edge_mlp0_body,
        grid=(nb,),
        in_specs=[
            pl.BlockSpec((_BE, _H), lambda i: (i, 0)),
            pl.BlockSpec((_H, _H), lambda i: (0, 0)),
            pl.BlockSpec((_H, _H), lambda i: (0, 0)),
        ],
        out_specs=pl.BlockSpec((_BE, _H), lambda i: (i, 0)),
        out_shape=jax.ShapeDtypeStruct((_EP, _H), jnp.float32),
    )(bsum, w1a, w1b)


def _edge_mlp_last_body(x1_ref, bsum_ref, w1a_ref, w1b_ref, bsrc_ref,
                        out_ref, g1_ref):
    z = _dot(x1_ref[...] + bsum_ref[...], w1a_ref[...])
    h = jax.nn.relu(_dot(jax.nn.relu(z), w1b_ref[...]))
    out_ref[...] = h
    onehot = (bsrc_ref[...] == jax.lax.broadcasted_iota(
        jnp.int32, (_BE, _G), 1)).astype(jnp.float32)

    @pl.when(pl.program_id(0) == 0)
    def _():
        g1_ref[...] = jnp.zeros_like(g1_ref)

    g1_ref[...] += _dotx(onehot, h, (((0,), (0,)), ((), ())))


def _edge_mlp_last(x1, bsum, w1a, w1b, bsrc2d):
    nb = _EP // _BE
    return pl.pallas_call(
        _edge_mlp_last_body,
        grid=(nb,),
        in_specs=[
            pl.BlockSpec((_BE, _H), lambda i: (i, 0)),
            pl.BlockSpec((_BE, _H), lambda i: (i, 0)),
            pl.BlockSpec((_H, _H), lambda i: (0, 0)),
            pl.BlockSpec((_H, _H), lambda i: (0, 0)),
            pl.BlockSpec((_BE, 1), lambda i: (i, 0)),
        ],
        out_specs=[
            pl.BlockSpec((_BE, _H), lambda i: (i, 0)),
            pl.BlockSpec((_G, _H), lambda i: (0, 0)),
        ],
        out_shape=[
            jax.ShapeDtypeStruct((_EP, _H), jnp.float32),
            jax.ShapeDtypeStruct((_G, _H), jnp.float32),
        ],
    )(x1, bsum, w1a, w1b, bsrc2d)


def _pe_common(fil128_ref, death_ref, r_ref, w0r_ref, w1r_ref, wp2e_ref,
               batch_ref):
    fil = fil128_ref[...][:, :_NF]
    death = death_ref[...]
    fil128 = _dot(fil, r_ref[...])
    death128 = _dot(death, r_ref[...])
    m = jax.nn.relu(fil128 * w0r_ref[...] + death128 * w1r_ref[...])
    pe = _dot(m, wp2e_ref[...])
    onehot = (batch_ref[...] == jax.lax.broadcasted_iota(
        jnp.int32, (_BN, _G), 1)).astype(jnp.float32)
    return pe, onehot


def _pe_body(fil16_ref, death_ref, r_ref, w0r_ref, w1r_ref, wp2e_ref,
             batch_ref, ph_ref):
    pe, onehot = _pe_common(fil16_ref, death_ref, r_ref, w0r_ref,
                            w1r_ref, wp2e_ref, batch_ref)

    @pl.when(pl.program_id(0) == 0)
    def _():
        ph_ref[...] = jnp.zeros_like(ph_ref)

    ph_ref[...] += _dotx(onehot, pe, (((0,), (0,)), ((), ())))


def _pe_last_body(fil16_ref, death_ref, r_ref, w0r_ref, w1r_ref,
                  wp2e_ref, batch_ref, x0_ref, ph_ref, g0_ref):
    pe, onehot = _pe_common(fil16_ref, death_ref, r_ref, w0r_ref,
                            w1r_ref, wp2e_ref, batch_ref)

    @pl.when(pl.program_id(0) == 0)
    def _():
        ph_ref[...] = jnp.zeros_like(ph_ref)
        g0_ref[...] = jnp.zeros_like(g0_ref)

    ph_ref[...] += _dotx(onehot, pe, (((0,), (0,)), ((), ())))
    g0_ref[...] += _dotx(onehot, x0_ref[...], (((0,), (0,)), ((), ())))


_pe_common_specs = [
    pl.BlockSpec((_BN, _H), lambda i: (i, 0)),
    pl.BlockSpec((_BN, _NF), lambda i: (i, 0)),
    pl.BlockSpec((_NF, _H), lambda i: (0, 0)),
    pl.BlockSpec((1, _H), lambda i: (0, 0)),
    pl.BlockSpec((1, _H), lambda i: (0, 0)),
    pl.BlockSpec((_H, _OPH), lambda i: (0, 0)),
    pl.BlockSpec((_BN, 1), lambda i: (i, 0)),
]


def _pe_readout(fil16, death, r, w0r, w1r, wp2e, batch2d):
    nb = _N // _BN
    return pl.pallas_call(
        _pe_body,
        grid=(nb,),
        in_specs=list(_pe_common_specs),
        out_specs=pl.BlockSpec((_G, _OPH), lambda i: (0, 0)),
        out_shape=jax.ShapeDtypeStruct((_G, _OPH), jnp.float32),
    )(fil16, death, r, w0r, w1r, wp2e, batch2d)


def _pe_readout_last(fil16, death, r, w0r, w1r, wp2e, batch2d, x0):
    nb = _N // _BN
    return pl.pallas_call(
        _pe_last_body,
        grid=(nb,),
        in_specs=list(_pe_common_specs) + [
            pl.BlockSpec((_BN, _H), lambda i: (i, 0)),
        ],
        out_specs=[
            pl.BlockSpec((_G, _OPH), lambda i: (0, 0)),
            pl.BlockSpec((_G, _H), lambda i: (0, 0)),
        ],
        out_shape=[
            jax.ShapeDtypeStruct((_G, _OPH), jnp.float32),
            jax.ShapeDtypeStruct((_G, _H), jnp.float32),
        ],
    )(fil16, death, r, w0r, w1r, wp2e, batch2d, x0)


def _head_body(g0_ref, g1_ref, ph0_ref, ph1_ref, ph2_ref, wl0_ref, b0_ref,
               wl1_ref, b1_ref, w2a_ref, w2b_ref, b2_ref, out_ref):
    z = (jax.nn.relu(_dot(g0_ref[...], wl0_ref[...]) + b0_ref[...]) +
         jax.nn.relu(_dot(g1_ref[...], wl1_ref[...]) + b1_ref[...]))
    ph = (ph0_ref[...] + ph1_ref[...] + ph2_ref[...]) * (1.0 / 3.0)
    out_ref[...] = _dot(z, w2a_ref[...]) + _dot(ph, w2b_ref[...]) + b2_ref[...]


def _head(g0, g1, ph0, ph1, ph2, wl0, b0, wl1, b1, w2a, w2b, b2):
    return pl.pallas_call(
        _head_body,
        out_shape=jax.ShapeDtypeStruct((_G, 1), jnp.float32),
    )(g0, g1, ph0, ph1, ph2, wl0, b0, wl1, b1, w2a, w2b, b2)


def kernel(atom_ids, edge_index, batch, atom_table, W0a, W0b, W1a, W1b, Wfil,
           Wp1, Wp2, Wlin1_0, b1_0, Wlin1_1, b1_1, Wlin2, blin2):
    src = edge_index[0]
    dst = edge_index[1]

    pad = _EP - _E
    spad = jnp.full((pad,), _N, jnp.int32)
    srcs = jnp.concatenate([src.astype(jnp.int32), spad])
    dsts = jnp.concatenate([dst.astype(jnp.int32), spad])
    srcs3 = srcs.reshape(_NW, _NCH, _C)
    dsts3 = dsts.reshape(_NW, _NCH, _C)
    srcs3c = srcs.reshape(_NW, _NCH2, _C2)
    dsts3c = dsts.reshape(_NW, _NCH2, _C2)
    dx8 = ((dsts[:, None] * _NF) +
           jnp.arange(_NF, dtype=jnp.int32)[None, :]).reshape(
               _NW, _NCH, _C * _NF)

    npad = _NPAD - _N
    ids2d = jnp.concatenate([atom_ids.astype(jnp.int32),
                             jnp.zeros((npad,), jnp.int32)]).reshape(_NPAD, 1)
    tbl_pad = jnp.zeros((_H, _H), jnp.float32).at[:119].set(atom_table)
    x0 = _embed_tc(ids2d, tbl_pad)
    batchp = jnp.concatenate([batch.astype(jnp.int32),
                              jnp.full((npad,), _G, jnp.int32)])
    batch2d = batchp.reshape(_NPAD, 1)
    bsrc2d = _sc_bsrc(batchp, srcs3).reshape(_EP, 1)

    eye8 = jnp.eye(_NF, dtype=jnp.float32)
    r = jnp.repeat(eye8, _FH, axis=1)
    w0r = jnp.tile(Wp1[:, 0, :], (1, _NF)).reshape(_L, 1, _NF * _FH)
    w1r = jnp.tile(Wp1[:, 1, :], (1, _NF)).reshape(_L, 1, _NF * _FH)
    wp2e = jnp.tile(Wp2, (1, _NF, 1)) / float(_NF)

    t32 = _sc_touched(dsts3).reshape(_NW, _NPAD // _H, _H)
    touched = _flat_max32(t32, _NPAD // _H, _NPAD // _H).reshape(_NPAD, 1)

    ph_list = []
    g0 = g1 = None
    x1 = None
    for l in range(_L):
        agg2, bsum = _sc_agg_bsum(x0, srcs3c, dsts3c)
        h, fil128 = _node_update(x0, agg2, W0a[l], W0b[l], Wfil[l])
        if l == 0:
            x1 = _edge_mlp0(bsum, W1a[l], W1b[l])
        elif l == _L - 1:
            x1, g1 = _edge_mlp_last(x1, bsum, W1a[l], W1b[l], bsrc2d)
        else:
            x1 = _edge_mlp(x1, bsum, W1a[l], W1b[l])
        x0 = h
        death32 = _sc_death(fil128, srcs3, dx8).reshape(
            _NW, _FLAT8 // _H, _H)
        dmax = _flat_max32(death32, _FLAT8 // _H, _H).reshape(_NPAD, _NF)
        death = _death_merge(dmax, fil128, touched)
        if l == _L - 1:
            ph_l, g0 = _pe_readout_last(fil128, death, r, w0r[l], w1r[l],
                                        wp2e[l], batch2d, x0)
        else:
            ph_l = _pe_readout(fil128, death, r, w0r[l], w1r[l], wp2e[l],
                               batch2d)
        ph_list.append(ph_l)

    b0_2d = b1_0.reshape(1, 2 * _H)
    b1_2d = b1_1.reshape(1, 2 * _H)
    w2a = Wlin2[:2 * _H]
    w2b = Wlin2[2 * _H:]
    b2_2d = blin2.reshape(1, 1)
    return _head(g0, g1, ph_list[0], ph_list[1], ph_list[2],
                 Wlin1_0, b0_2d, Wlin1_1, b1_2d, w2a, w2b, b2_2d)


# speedup vs baseline: 3.0672x; 1.0006x over previous
import functools

import jax
import jax.numpy as jnp
from jax import lax
from jax.experimental import pallas as pl
from jax.experimental.pallas import tpu as pltpu
from jax.experimental.pallas import tpu_sc as plsc

_PREC = jax.lax.Precision.HIGHEST

_N = 10000
_E = 320000
_H = 128
_G = 256
_L = 3
_NF = 8
_FH = 16
_OPH = 64

_BN = 2000

_NW = 32
_EP = 327680
_CW = _EP // _NW
_C = 128
_NCH = _CW // _C

_BE = 4096


def _sc_mesh():
    return plsc.VectorSubcoreMesh(core_axis_name="c", subcore_axis_name="s")


def _wid():
    return lax.axis_index("s") * 2 + lax.axis_index("c")


_NPAD = 10240
_RPT = _NPAD // 16
_FLAT8 = _NPAD * _NF
_RFT = _FLAT8 // 16


_C2 = 64
_NCH2 = _CW // _C2
_HCH2 = _NCH2 // 4


def _agg_bsum_body(table, src3, dst3, agg_out, bsum_out, idx_s, idx_d,
                   ra, rb, acc, gs0, gs1, ws0, ws1):
    w = _wid()
    c = lax.axis_index("c")
    t = lax.axis_index("s")

    def zrow(r, c2):
        for j in range(8):
            ra[0, r, pl.ds(j * 16, 16)] = jnp.zeros((16,), jnp.float32)
        return c2

    lax.fori_loop(0, _C2, zrow, 0)
    for k in range(_RPT // _C2):
        pltpu.sync_copy(ra.at[0], acc.at[pl.ds(t * _RPT + k * _C2, _C2)])
    plsc.subcore_barrier()

    gsems = (gs0, gs1)
    wsems = (ws0, ws1)

    for half in range(4):
        pltpu.sync_copy(src3.at[w, pl.ds(half * _HCH2, _HCH2)], idx_s)
        pltpu.sync_copy(dst3.at[w, pl.ds(half * _HCH2, _HCH2)], idx_d)

        def fetch(i, b):
            pltpu.make_async_copy(table.at[idx_s.at[i]], ra.at[b],
                                  gsems[b]).start()
            pltpu.make_async_copy(table.at[idx_d.at[i]], rb.at[b],
                                  gsems[b]).start()

        fetch(0, 0)
        fetch(1, 1)
        base0 = w * _CW + half * _HCH2 * _C2

        def outer(it, carry):
            i0 = it * 2
            for b in range(2):
                i = i0 + b
                pltpu.make_async_copy(table.at[idx_s.at[i]], ra.at[b],
                                      gsems[b]).wait()
                pltpu.make_async_copy(table.at[idx_d.at[i]], rb.at[b],
                                      gsems[b]).wait()
                rab, rbb = ra.at[b], rb.at[b]

                def add_row(rr, c2):
                    for j in range(8):
                        sl = pl.ds(j * 16, 16)
                        rbb[rr, sl] = rab[rr, sl] + rbb[rr, sl]
                    return c2

                lax.fori_loop(0, _C2, add_row, 0, unroll=2)
                pltpu.make_async_copy(
                    rb.at[b], bsum_out.at[pl.ds(base0 + i * _C2, _C2)],
                    wsems[b]).start()
                pltpu.sync_copy(ra.at[b], acc.at[idx_d.at[i]], add=True)

                @pl.when(i + 2 < _HCH2)
                def _():
                    pltpu.make_async_copy(
                        rb.at[b], bsum_out.at[pl.ds(0, _C2)], wsems[b]).wait()
                    fetch(i + 2, b)
            return carry

        lax.fori_loop(0, _HCH2 // 2, outer, 0)
        for b in range(2):
            pltpu.make_async_copy(rb.at[b], bsum_out.at[pl.ds(0, _C2)],
                                  wsems[b]).wait()

    plsc.subcore_barrier()
    pltpu.sync_copy(acc.at[pl.ds(t * _RPT, _RPT)],
                    agg_out.at[c, pl.ds(t * _RPT, _RPT)])


def _sc_agg_bsum(table, srcs3c, dsts3c):
    return pl.kernel(
        _agg_bsum_body,
        mesh=_sc_mesh(),
        out_type=[
            jax.ShapeDtypeStruct((2, _NPAD, _H), jnp.float32),
            jax.ShapeDtypeStruct((_EP, _H), jnp.float32),
        ],
        scratch_types=[
            pltpu.VMEM((_HCH2, _C2), jnp.int32),
            pltpu.VMEM((_HCH2, _C2), jnp.int32),
            pltpu.VMEM((2, _C2, _H), jnp.float32),
            pltpu.VMEM((2, _C2, _H), jnp.float32),
            pltpu.VMEM_SHARED((_NPAD, _H), jnp.float32),
            pltpu.SemaphoreType.DMA,
            pltpu.SemaphoreType.DMA,
            pltpu.SemaphoreType.DMA,
            pltpu.SemaphoreType.DMA,
        ],
    )(table, srcs3c, dsts3c)


def _bsrc_body(batchp, src3, out, batchv, idx_s, ob0, ob1, ws0, ws1):
    w = _wid()
    pltpu.sync_copy(batchp, batchv)
    pltpu.sync_copy(src3.at[w], idx_s)
    obs = (ob0, ob1)
    wsems = (ws0, ws1)

    def run(i, b):
        for j in range(_C // 16):
            sl = pl.ds(j * 16, 16)
            sidx = idx_s[i, sl]
            obs[b][sl] = plsc.load_gather(batchv, [sidx])
        pltpu.make_async_copy(obs[b], out.at[pl.ds(w * _CW + i * _C, _C)],
                              wsems[b]).start()

    def outer(it, carry):
        i0 = it * 2
        for b in range(2):
            i = i0 + b

            @pl.when(i >= 2)
            def _():
                pltpu.make_async_copy(obs[b], out.at[pl.ds(0, _C)],
                                      wsems[b]).wait()

            run(i, b)
        return carry

    lax.fori_loop(0, _NCH // 2, outer, 0)
    for b in range(2):
        pltpu.make_async_copy(obs[b], out.at[pl.ds(0, _C)], wsems[b]).wait()


def _sc_bsrc(batchp, srcs3):
    return pl.kernel(
        _bsrc_body,
        mesh=_sc_mesh(),
        compiler_params=pltpu.CompilerParams(needs_layout_passes=False),
        out_type=jax.ShapeDtypeStruct((_EP,), jnp.int32),
        scratch_types=[
            pltpu.VMEM((_NPAD,), jnp.int32),
            pltpu.VMEM((_NCH, _C), jnp.int32),
            pltpu.VMEM((_C,), jnp.int32),
            pltpu.VMEM((_C,), jnp.int32),
            pltpu.SemaphoreType.DMA,
            pltpu.SemaphoreType.DMA,
        ],
    )(batchp, srcs3)


def _death_body(fil128, src3, dx8, out, local, idx_s, dx0, dx1, rs0, rs1,
                gs0, gs1):
    w = _wid()
    pltpu.sync_copy(src3.at[w], idx_s)

    def zslot(k, c2):
        local[pl.ds(k * 16, 16)] = jnp.zeros((16,), jnp.float32)
        return c2

    lax.fori_loop(0, _FLAT8 // 16, zslot, 0)

    iota = jax.lax.iota(jnp.int32, 16)
    mlo = iota < 8
    mhi = iota >= 8
    gsems = (gs0, gs1)
    dxs = (dx0, dx1)
    rss = (rs0, rs1)

    def fetch(i, b):
        pltpu.make_async_copy(fil128.at[idx_s.at[i]], rss[b],
                              gsems[b]).start()
        pltpu.make_async_copy(dx8.at[w, i], dxs[b], gsems[b]).start()

    fetch(0, 0)
    fetch(1, 1)

    def outer(it, carry):
        i0 = it * 2
        for b in range(2):
            i = i0 + b
            pltpu.make_async_copy(fil128.at[idx_s.at[i]], rss[b],
                                  gsems[b]).wait()
            pltpu.make_async_copy(dx8.at[w, i], dxs[b], gsems[b]).wait()
            rsb, dxb = rss[b], dxs[b]

            def pair(r, c2):
                r2 = r * 2
                e0 = rsb[r2, pl.ds(0, 16)]
                e1 = rsb[r2 + 1, pl.ds(0, 16)]
                idx = dxb[pl.ds(r2 * 8, 16)]
                old = plsc.load_gather(local, [idx])
                plsc.store_scatter(local, [idx], jnp.maximum(old, e0),
                                   mask=mlo)
                old2 = plsc.load_gather(local, [idx])
                plsc.store_scatter(local, [idx], jnp.maximum(old2, e1),
                                   mask=mhi)
                return c2

            lax.fori_loop(0, _C // 2, pair, 0, unroll=4)

            @pl.when(i + 2 < _NCH)
            def _():
                fetch(i + 2, b)
        return carry

    lax.fori_loop(0, _NCH // 2, outer, 0)
    pltpu.sync_copy(local, out.at[w])


def _sc_death(fil128, src3, dx8):
    return pl.kernel(
        _death_body,
        mesh=_sc_mesh(),
        compiler_params=pltpu.CompilerParams(needs_layout_passes=False),
        out_type=jax.ShapeDtypeStruct((_NW, _FLAT8), jnp.float32),
        scratch_types=[
            pltpu.VMEM((_FLAT8,), jnp.float32),
            pltpu.VMEM((_NCH, _C), jnp.int32),
            pltpu.VMEM((_C * _NF,), jnp.int32),
            pltpu.VMEM((_C * _NF,), jnp.int32),
            pltpu.VMEM((_C, _H), jnp.float32),
            pltpu.VMEM((_C, _H), jnp.float32),
            pltpu.SemaphoreType.DMA,
            pltpu.SemaphoreType.DMA,
        ],
    )(fil128, src3, dx8)


def _touched_body(dst3, out, local, idx_d):
    w = _wid()
    pltpu.sync_copy(dst3.at[w], idx_d)

    def zslot(k, c2):
        local[pl.ds(k * 16, 16)] = jnp.zeros((16,), jnp.float32)
        return c2

    lax.fori_loop(0, _NPAD // 16, zslot, 0)
    ones = jnp.ones((16,), jnp.float32)

    def chunk(i, c2):
        for j in range(_C // 16):
            idx = idx_d[i, pl.ds(j * 16, 16)]
            plsc.store_scatter(local, [idx], ones)
        return c2

    lax.fori_loop(0, _NCH, chunk, 0)
    pltpu.sync_copy(local, out.at[w])


def _sc_touched(dsts3):
    return pl.kernel(
        _touched_body,
        mesh=_sc_mesh(),
        compiler_params=pltpu.CompilerParams(needs_layout_passes=False),
        out_type=jax.ShapeDtypeStruct((_NW, _NPAD), jnp.float32),
        scratch_types=[
            pltpu.VMEM((_NPAD,), jnp.float32),
            pltpu.VMEM((_NCH, _C), jnp.int32),
        ],
    )(dsts3)


def _fmax_body(*refs):
    out_ref = refs[-1]
    d = refs[0][0]
    for k in range(1, _NW):
        d = jnp.maximum(d, refs[k][0])
    out_ref[...] = d


def _flat_max32(a32r, rrows, brows):
    nb = rrows // brows
    return pl.pallas_call(
        _fmax_body,
        grid=(nb,),
        in_specs=[pl.BlockSpec((1, brows, _H), functools.partial(
            lambda k, i: (k, i, 0), k)) for k in range(_NW)],
        out_specs=pl.BlockSpec((brows, _H), lambda i: (i, 0)),
        out_shape=jax.ShapeDtypeStruct((rrows, _H), jnp.float32),
    )(*([a32r] * _NW))


def _death_merge_body(d_ref, fil128_ref, tch_ref, out_ref):
    fil = fil128_ref[...][:, :_NF]
    out_ref[...] = jnp.maximum(d_ref[...], fil * tch_ref[...])


def _death_merge(dmax, fil128, touched):
    nb = _N // _BN
    return pl.pallas_call(
        _death_merge_body,
        grid=(nb,),
        in_specs=[
            pl.BlockSpec((_BN, _NF), lambda i: (i, 0)),
            pl.BlockSpec((_BN, _H), lambda i: (i, 0)),
            pl.BlockSpec((_BN, 1), lambda i: (i, 0)),
        ],
        out_specs=pl.BlockSpec((_BN, _NF), lambda i: (i, 0)),
        out_shape=jax.ShapeDtypeStruct((_N, _NF), jnp.float32),
    )(dmax, fil128, touched)


def _dot(a, b, dims=None, prec=None):
    prec = prec or jax.lax.Precision.DEFAULT
    if dims is None:
        return jax.lax.dot(a, b, precision=prec,
                           preferred_element_type=jnp.float32)
    return jax.lax.dot_general(a, b, dims, precision=prec,
                               preferred_element_type=jnp.float32)


def _dotx(a, b, dims=None):
    return _dot(a, b, dims, prec=jax.lax.Precision.HIGHEST)


_BN2 = 2048


def _embed_body(ids_ref, tbl_ref, out_ref):
    onehot = (ids_ref[...] == jax.lax.broadcasted_iota(
        jnp.int32, (_BN2, _H), 1)).astype(jnp.float32)
    out_ref[...] = _dotx(onehot, tbl_ref[...])


def _embed_tc(ids2d, tbl_pad):
    nb = _NPAD // _BN2
    return pl.pallas_call(
        _embed_body,
        grid=(nb,),
        in_specs=[
            pl.BlockSpec((_BN2, 1), lambda i: (i, 0)),
            pl.BlockSpec((_H, _H), lambda i: (0, 0)),
        ],
        out_specs=pl.BlockSpec((_BN2, _H), lambda i: (i, 0)),
        out_shape=jax.ShapeDtypeStruct((_NPAD, _H), jnp.float32),
    )(ids2d, tbl_pad)


def _node_update_body(x0_ref, a0_ref, a1_ref, w0a_ref, w0b_ref,
                      wfil_ref, h_ref, fil_ref):
    x0 = x0_ref[...]
    agg = a0_ref[0] + a1_ref[0]
    t = jax.nn.relu(_dot(x0 + agg, w0a_ref[...]))
    h = jax.nn.relu(_dot(t, w0b_ref[...]))
    h_ref[...] = h
    fil = jax.nn.sigmoid(_dot(h, wfil_ref[...]))
    fil_ref[...] = jnp.tile(fil, (1, 16))


def _node_update(x0, agg2, w0a, w0b, wfil):
    nb = _NPAD // _BN2
    return pl.pallas_call(
        _node_update_body,
        grid=(nb,),
        in_specs=[
            pl.BlockSpec((_BN2, _H), lambda i: (i, 0)),
            pl.BlockSpec((1, _BN2, _H), lambda i: (0, i, 0)),
            pl.BlockSpec((1, _BN2, _H), lambda i: (1, i, 0)),
            pl.BlockSpec((_H, _H), lambda i: (0, 0)),
            pl.BlockSpec((_H, _H), lambda i: (0, 0)),
            pl.BlockSpec((_H, _NF), lambda i: (0, 0)),
        ],
        out_specs=[
            pl.BlockSpec((_BN2, _H), lambda i: (i, 0)),
            pl.BlockSpec((_BN2, _H), lambda i: (i, 0)),
        ],
        out_shape=[
            jax.ShapeDtypeStruct((_NPAD, _H), jnp.float32),
            jax.ShapeDtypeStruct((_NPAD, _H), jnp.float32),
        ],
    )(x0, agg2, agg2, w0a, w0b, wfil)


def _edge_mlp_body(x1_ref, bsum_ref, w1a_ref, w1b_ref, out_ref):
    z = _dot(x1_ref[...] + bsum_ref[...], w1a_ref[...])
    out_ref[...] = jax.nn.relu(_dot(jax.nn.relu(z), w1b_ref[...]))


def _edge_mlp(x1, bsum, w1a, w1b):
    nb = _EP // _BE
    return pl.pallas_call(
        _edge_mlp_body,
        grid=(nb,),
        in_specs=[
            pl.BlockSpec((_BE, _H), lambda i: (i, 0)),
            pl.BlockSpec((_BE, _H), lambda i: (i, 0)),
            pl.BlockSpec((_H, _H), lambda i: (0, 0)),
            pl.BlockSpec((_H, _H), lambda i: (0, 0)),
        ],
        out_specs=pl.BlockSpec((_BE, _H), lambda i: (i, 0)),
        out_shape=jax.ShapeDtypeStruct((_EP, _H), jnp.float32),
    )(x1, bsum, w1a, w1b)


def _edge_mlp0_body(bsum_ref, w1a_ref, w1b_ref, out_ref):
    z = _dot(bsum_ref[...] * 2.0, w1a_ref[...])
    out_ref[...] = jax.nn.relu(_dot(jax.nn.relu(z), w1b_ref[...]))


def _edge_mlp0(bsum, w1a, w1b):
    nb = _EP // _BE
    return pl.pallas_call(
        _edge_mlp0_body,
        grid=(nb,),
        in_specs=[
            pl.BlockSpec((_BE, _H), lambda i: (i, 0)),
            pl.BlockSpec((_H, _H), lambda i: (0, 0)),
            pl.BlockSpec((_H, _H), lambda i: (0, 0)),
        ],
        out_specs=pl.BlockSpec((_BE, _H), lambda i: (i, 0)),
        out_shape=jax.ShapeDtypeStruct((_EP, _H), jnp.float32),
    )(bsum, w1a, w1b)


def _edge_mlp_last_body(x1_ref, bsum_ref, w1a_ref, w1b_ref, bsrc_ref,
                        out_ref, g1_ref):
    z = _dot(x1_ref[...] + bsum_ref[...], w1a_ref[...])
    h = jax.nn.relu(_dot(jax.nn.relu(z), w1b_ref[...]))
    out_ref[...] = h
    onehot = (bsrc_ref[...] == jax.lax.broadcasted_iota(
        jnp.int32, (_BE, _G), 1)).astype(jnp.float32)

    @pl.when(pl.program_id(0) == 0)
    def _():
        g1_ref[...] = jnp.zeros_like(g1_ref)

    g1_ref[...] += _dotx(onehot, h, (((0,), (0,)), ((), ())))


def _edge_mlp_last(x1, bsum, w1a, w1b, bsrc2d):
    nb = _EP // _BE
    return pl.pallas_call(
        _edge_mlp_last_body,
        grid=(nb,),
        in_specs=[
            pl.BlockSpec((_BE, _H), lambda i: (i, 0)),
            pl.BlockSpec((_BE, _H), lambda i: (i, 0)),
            pl.BlockSpec((_H, _H), lambda i: (0, 0)),
            pl.BlockSpec((_H, _H), lambda i: (0, 0)),
            pl.BlockSpec((_BE, 1), lambda i: (i, 0)),
        ],
        out_specs=[
            pl.BlockSpec((_BE, _H), lambda i: (i, 0)),
            pl.BlockSpec((_G, _H), lambda i: (0, 0)),
        ],
        out_shape=[
            jax.ShapeDtypeStruct((_EP, _H), jnp.float32),
            jax.ShapeDtypeStruct((_G, _H), jnp.float32),
        ],
    )(x1, bsum, w1a, w1b, bsrc2d)


def _pe_common(fil128_ref, death_ref, r_ref, w0r_ref, w1r_ref, wp2e_ref,
               batch_ref):
    fil = fil128_ref[...][:, :_NF]
    death = death_ref[...]
    fil128 = _dot(fil, r_ref[...])
    death128 = _dot(death, r_ref[...])
    m = jax.nn.relu(fil128 * w0r_ref[...] + death128 * w1r_ref[...])
    pe = _dot(m, wp2e_ref[...])
    onehot = (batch_ref[...] == jax.lax.broadcasted_iota(
        jnp.int32, (_BN, _G), 1)).astype(jnp.float32)
    return pe, onehot


def _pe_body(fil16_ref, death_ref, r_ref, w0r_ref, w1r_ref, wp2e_ref,
             batch_ref, ph_ref):
    pe, onehot = _pe_common(fil16_ref, death_ref, r_ref, w0r_ref,
                            w1r_ref, wp2e_ref, batch_ref)

    @pl.when(pl.program_id(0) == 0)
    def _():
        ph_ref[...] = jnp.zeros_like(ph_ref)

    ph_ref[...] += _dotx(onehot, pe, (((0,), (0,)), ((), ())))


def _pe_last_body(fil16_ref, death_ref, r_ref, w0r_ref, w1r_ref,
                  wp2e_ref, batch_ref, x0_ref, ph_ref, g0_ref):
    pe, onehot = _pe_common(fil16_ref, death_ref, r_ref, w0r_ref,
                            w1r_ref, wp2e_ref, batch_ref)

    @pl.when(pl.program_id(0) == 0)
    def _():
        ph_ref[...] = jnp.zeros_like(ph_ref)
        g0_ref[...] = jnp.zeros_like(g0_ref)

    ph_ref[...] += _dotx(onehot, pe, (((0,), (0,)), ((), ())))
    g0_ref[...] += _dotx(onehot, x0_ref[...], (((0,), (0,)), ((), ())))


_pe_common_specs = [
    pl.BlockSpec((_BN, _H), lambda i: (i, 0)),
    pl.BlockSpec((_BN, _NF), lambda i: (i, 0)),
    pl.BlockSpec((_NF, _H), lambda i: (0, 0)),
    pl.BlockSpec((1, _H), lambda i: (0, 0)),
    pl.BlockSpec((1, _H), lambda i: (0, 0)),
    pl.BlockSpec((_H, _OPH), lambda i: (0, 0)),
    pl.BlockSpec((_BN, 1), lambda i: (i, 0)),
]


def _pe_readout(fil16, death, r, w0r, w1r, wp2e, batch2d):
    nb = _N // _BN
    return pl.pallas_call(
        _pe_body,
        grid=(nb,),
        in_specs=list(_pe_common_specs),
        out_specs=pl.BlockSpec((_G, _OPH), lambda i: (0, 0)),
        out_shape=jax.ShapeDtypeStruct((_G, _OPH), jnp.float32),
    )(fil16, death, r, w0r, w1r, wp2e, batch2d)


def _pe_readout_last(fil16, death, r, w0r, w1r, wp2e, batch2d, x0):
    nb = _N // _BN
    return pl.pallas_call(
        _pe_last_body,
        grid=(nb,),
        in_specs=list(_pe_common_specs) + [
            pl.BlockSpec((_BN, _H), lambda i: (i, 0)),
        ],
        out_specs=[
            pl.BlockSpec((_G, _OPH), lambda i: (0, 0)),
            pl.BlockSpec((_G, _H), lambda i: (0, 0)),
        ],
        out_shape=[
            jax.ShapeDtypeStruct((_G, _OPH), jnp.float32),
            jax.ShapeDtypeStruct((_G, _H), jnp.float32),
        ],
    )(fil16, death, r, w0r, w1r, wp2e, batch2d, x0)


def _head_body(g0_ref, g1_ref, ph0_ref, ph1_ref, ph2_ref, wl0_ref, b0_ref,
               wl1_ref, b1_ref, w2a_ref, w2b_ref, b2_ref, out_ref):
    z = (jax.nn.relu(_dot(g0_ref[...], wl0_ref[...]) + b0_ref[...]) +
         jax.nn.relu(_dot(g1_ref[...], wl1_ref[...]) + b1_ref[...]))
    ph = (ph0_ref[...] + ph1_ref[...] + ph2_ref[...]) * (1.0 / 3.0)
    out_ref[...] = _dot(z, w2a_ref[...]) + _dot(ph, w2b_ref[...]) + b2_ref[...]


def _head(g0, g1, ph0, ph1, ph2, wl0, b0, wl1, b1, w2a, w2b, b2):
    return pl.pallas_call(
        _head_body,
        out_shape=jax.ShapeDtypeStruct((_G, 1), jnp.float32),
    )(g0, g1, ph0, ph1, ph2, wl0, b0, wl1, b1, w2a, w2b, b2)


def kernel(atom_ids, edge_index, batch, atom_table, W0a, W0b, W1a, W1b, Wfil,
           Wp1, Wp2, Wlin1_0, b1_0, Wlin1_1, b1_1, Wlin2, blin2):
    src = edge_index[0]
    dst = edge_index[1]

    pad = _EP - _E
    spad = jnp.full((pad,), _N, jnp.int32)
    srcs = jnp.concatenate([src.astype(jnp.int32), spad])
    dsts = jnp.concatenate([dst.astype(jnp.int32), spad])
    srcs3 = srcs.reshape(_NW, _NCH, _C)
    dsts3 = dsts.reshape(_NW, _NCH, _C)
    srcs3c = srcs.reshape(_NW, _NCH2, _C2)
    dsts3c = dsts.reshape(_NW, _NCH2, _C2)
    dx8 = ((dsts[:, None] * _NF) +
           jnp.arange(_NF, dtype=jnp.int32)[None, :]).reshape(
               _NW, _NCH, _C * _NF)

    npad = _NPAD - _N
    ids2d = jnp.concatenate([atom_ids.astype(jnp.int32),
                             jnp.zeros((npad,), jnp.int32)]).reshape(_NPAD, 1)
    tbl_pad = jnp.zeros((_H, _H), jnp.float32).at[:119].set(atom_table)
    x0 = _embed_tc(ids2d, tbl_pad)
    batchp = jnp.concatenate([batch.astype(jnp.int32),
                              jnp.full((npad,), _G, jnp.int32)])
    batch2d = batchp.reshape(_NPAD, 1)
    bsrc2d = _sc_bsrc(batchp, srcs3).reshape(_EP, 1)

    eye8 = jnp.eye(_NF, dtype=jnp.float32)
    r = jnp.repeat(eye8, _FH, axis=1)
    w0r = jnp.tile(Wp1[:, 0, :], (1, _NF)).reshape(_L, 1, _NF * _FH)
    w1r = jnp.tile(Wp1[:, 1, :], (1, _NF)).reshape(_L, 1, _NF * _FH)
    wp2e = jnp.tile(Wp2, (1, _NF, 1)) / float(_NF)

    t32 = _sc_touched(dsts3).reshape(_NW, _NPAD // _H, _H)
    touched = _flat_max32(t32, _NPAD // _H, _NPAD // _H).reshape(_NPAD, 1)

    ph_list = []
    g0 = g1 = None
    x1 = None
    for l in range(_L):
        agg2, bsum = _sc_agg_bsum(x0, srcs3c, dsts3c)
        h, fil128 = _node_update(x0, agg2, W0a[l], W0b[l], Wfil[l])
        if l == 0:
            x1 = _edge_mlp0(bsum, W1a[l], W1b[l])
        elif l == _L - 1:
            x1, g1 = _edge_mlp_last(x1, bsum, W1a[l], W1b[l], bsrc2d)
        else:
            x1 = _edge_mlp(x1, bsum, W1a[l], W1b[l])
        x0 = h
        death32 = _sc_death(fil128, srcs3, dx8).reshape(
            _NW, _FLAT8 // _H, _H)
        dmax = _flat_max32(death32, _FLAT8 // _H, _H).reshape(_NPAD, _NF)
        death = _death_merge(dmax, fil128, touched)
        if l == _L - 1:
            ph_l, g0 = _pe_readout_last(fil128, death, r, w0r[l], w1r[l],
                                        wp2e[l], batch2d, x0)
        else:
            ph_l = _pe_readout(fil128, death, r, w0r[l], w1r[l], wp2e[l],
                               batch2d)
        ph_list.append(ph_l)

    b0_2d = b1_0.reshape(1, 2 * _H)
    b1_2d = b1_1.reshape(1, 2 * _H)
    w2a = Wlin2[:2 * _H]
    w2b = Wlin2[2 * _H:]
    b2_2d = blin2.reshape(1, 1)
    return _head(g0, g1, ph_list[0], ph_list[1], ph_list[2],
                 Wlin1_0, b0_2d, Wlin1_1, b1_2d, w2a, w2b, b2_2d)


# speedup vs baseline: 3.0681x; 1.0003x over previous
import functools

import jax
import jax.numpy as jnp
from jax import lax
from jax.experimental import pallas as pl
from jax.experimental.pallas import tpu as pltpu
from jax.experimental.pallas import tpu_sc as plsc

_N = 10000
_E = 320000
_H = 128
_G = 256
_L = 3
_NF = 8
_FH = 16
_OPH = 64

_BN = 2000

_NW = 32
_EP = 327680
_CW = _EP // _NW
_C = 128
_NCH = _CW // _C

_BE = 4096


def _sc_mesh():
    return plsc.VectorSubcoreMesh(core_axis_name="c", subcore_axis_name="s")


def _wid():
    return lax.axis_index("s") * 2 + lax.axis_index("c")


_NPAD = 10240
_RPT = _NPAD // 16
_FLAT8 = _NPAD * _NF


_C2 = 64
_NCH2 = _CW // _C2
_HCH2 = _NCH2 // 4


def _agg_bsum_body(table, src3, dst3, agg_out, bsum_out, idx_s, idx_d,
                   ra, rb, acc, gs0, gs1, ws0, ws1):
    w = _wid()
    c = lax.axis_index("c")
    t = lax.axis_index("s")

    def zrow(r, c2):
        for j in range(8):
            ra[0, r, pl.ds(j * 16, 16)] = jnp.zeros((16,), jnp.float32)
        return c2

    lax.fori_loop(0, _C2, zrow, 0)
    for k in range(_RPT // _C2):
        pltpu.sync_copy(ra.at[0], acc.at[pl.ds(t * _RPT + k * _C2, _C2)])
    plsc.subcore_barrier()

    gsems = (gs0, gs1)
    wsems = (ws0, ws1)

    for half in range(4):
        pltpu.sync_copy(src3.at[w, pl.ds(half * _HCH2, _HCH2)], idx_s)
        pltpu.sync_copy(dst3.at[w, pl.ds(half * _HCH2, _HCH2)], idx_d)

        def fetch(i, b):
            pltpu.make_async_copy(table.at[idx_s.at[i]], ra.at[b],
                                  gsems[b]).start()
            pltpu.make_async_copy(table.at[idx_d.at[i]], rb.at[b],
                                  gsems[b]).start()

        fetch(0, 0)
        fetch(1, 1)
        base0 = w * _CW + half * _HCH2 * _C2

        def outer(it, carry):
            i0 = it * 2
            for b in range(2):
                i = i0 + b
                pltpu.make_async_copy(table.at[idx_s.at[i]], ra.at[b],
                                      gsems[b]).wait()
                pltpu.make_async_copy(table.at[idx_d.at[i]], rb.at[b],
                                      gsems[b]).wait()
                rab, rbb = ra.at[b], rb.at[b]

                def add_row(rr, c2):
                    for j in range(8):
                        sl = pl.ds(j * 16, 16)
                        rbb[rr, sl] = rab[rr, sl] + rbb[rr, sl]
                    return c2

                lax.fori_loop(0, _C2, add_row, 0, unroll=2)
                pltpu.make_async_copy(
                    rb.at[b], bsum_out.at[pl.ds(base0 + i * _C2, _C2)],
                    wsems[b]).start()
                pltpu.sync_copy(ra.at[b], acc.at[idx_d.at[i]], add=True)

                @pl.when(i + 2 < _HCH2)
                def _():
                    pltpu.make_async_copy(
                        rb.at[b], bsum_out.at[pl.ds(0, _C2)], wsems[b]).wait()
                    fetch(i + 2, b)
            return carry

        lax.fori_loop(0, _HCH2 // 2, outer, 0)
        for b in range(2):
            pltpu.make_async_copy(rb.at[b], bsum_out.at[pl.ds(0, _C2)],
                                  wsems[b]).wait()

    plsc.subcore_barrier()
    pltpu.sync_copy(acc.at[pl.ds(t * _RPT, _RPT)],
                    agg_out.at[c, pl.ds(t * _RPT, _RPT)])


def _sc_agg_bsum(table, srcs3c, dsts3c):
    return pl.kernel(
        _agg_bsum_body,
        mesh=_sc_mesh(),
        out_type=[
            jax.ShapeDtypeStruct((2, _NPAD, _H), jnp.float32),
            jax.ShapeDtypeStruct((_EP, _H), jnp.float32),
        ],
        scratch_types=[
            pltpu.VMEM((_HCH2, _C2), jnp.int32),
            pltpu.VMEM((_HCH2, _C2), jnp.int32),
            pltpu.VMEM((2, _C2, _H), jnp.float32),
            pltpu.VMEM((2, _C2, _H), jnp.float32),
            pltpu.VMEM_SHARED((_NPAD, _H), jnp.float32),
            pltpu.SemaphoreType.DMA,
            pltpu.SemaphoreType.DMA,
            pltpu.SemaphoreType.DMA,
            pltpu.SemaphoreType.DMA,
        ],
    )(table, srcs3c, dsts3c)


def _bsrc_body(batchp, src3, out, batchv, idx_s, ob0, ob1, ws0, ws1):
    w = _wid()
    pltpu.sync_copy(batchp, batchv)
    pltpu.sync_copy(src3.at[w], idx_s)
    obs = (ob0, ob1)
    wsems = (ws0, ws1)

    def run(i, b):
        for j in range(_C // 16):
            sl = pl.ds(j * 16, 16)
            sidx = idx_s[i, sl]
            obs[b][sl] = plsc.load_gather(batchv, [sidx])
        pltpu.make_async_copy(obs[b], out.at[pl.ds(w * _CW + i * _C, _C)],
                              wsems[b]).start()

    def outer(it, carry):
        i0 = it * 2
        for b in range(2):
            i = i0 + b

            @pl.when(i >= 2)
            def _():
                pltpu.make_async_copy(obs[b], out.at[pl.ds(0, _C)],
                                      wsems[b]).wait()

            run(i, b)
        return carry

    lax.fori_loop(0, _NCH // 2, outer, 0)
    for b in range(2):
        pltpu.make_async_copy(obs[b], out.at[pl.ds(0, _C)], wsems[b]).wait()


def _sc_bsrc(batchp, srcs3):
    return pl.kernel(
        _bsrc_body,
        mesh=_sc_mesh(),
        compiler_params=pltpu.CompilerParams(needs_layout_passes=False),
        out_type=jax.ShapeDtypeStruct((_EP,), jnp.int32),
        scratch_types=[
            pltpu.VMEM((_NPAD,), jnp.int32),
            pltpu.VMEM((_NCH, _C), jnp.int32),
            pltpu.VMEM((_C,), jnp.int32),
            pltpu.VMEM((_C,), jnp.int32),
            pltpu.SemaphoreType.DMA,
            pltpu.SemaphoreType.DMA,
        ],
    )(batchp, srcs3)


def _death_body(fil128, src3, dx8, out, local, idx_s, dx0, dx1, rs0, rs1,
                gs0, gs1):
    w = _wid()
    pltpu.sync_copy(src3.at[w], idx_s)

    def zslot(k, c2):
        local[pl.ds(k * 16, 16)] = jnp.zeros((16,), jnp.float32)
        return c2

    lax.fori_loop(0, _FLAT8 // 16, zslot, 0)

    iota = jax.lax.iota(jnp.int32, 16)
    mlo = iota < 8
    mhi = iota >= 8
    gsems = (gs0, gs1)
    dxs = (dx0, dx1)
    rss = (rs0, rs1)

    def fetch(i, b):
        pltpu.make_async_copy(fil128.at[idx_s.at[i]], rss[b],
                              gsems[b]).start()
        pltpu.make_async_copy(dx8.at[w, i], dxs[b], gsems[b]).start()

    fetch(0, 0)
    fetch(1, 1)

    def outer(it, carry):
        i0 = it * 2
        for b in range(2):
            i = i0 + b
            pltpu.make_async_copy(fil128.at[idx_s.at[i]], rss[b],
                                  gsems[b]).wait()
            pltpu.make_async_copy(dx8.at[w, i], dxs[b], gsems[b]).wait()
            rsb, dxb = rss[b], dxs[b]

            def pair(r, c2):
                r2 = r * 2
                e0 = rsb[r2, pl.ds(0, 16)]
                e1 = rsb[r2 + 1, pl.ds(0, 16)]
                idx = dxb[pl.ds(r2 * 8, 16)]
                old = plsc.load_gather(local, [idx])
                plsc.store_scatter(local, [idx], jnp.maximum(old, e0),
                                   mask=mlo)
                old2 = plsc.load_gather(local, [idx])
                plsc.store_scatter(local, [idx], jnp.maximum(old2, e1),
                                   mask=mhi)
                return c2

            lax.fori_loop(0, _C // 2, pair, 0, unroll=4)

            @pl.when(i + 2 < _NCH)
            def _():
                fetch(i + 2, b)
        return carry

    lax.fori_loop(0, _NCH // 2, outer, 0)
    pltpu.sync_copy(local, out.at[w])


def _sc_death(fil128, src3, dx8):
    return pl.kernel(
        _death_body,
        mesh=_sc_mesh(),
        compiler_params=pltpu.CompilerParams(needs_layout_passes=False),
        out_type=jax.ShapeDtypeStruct((_NW, _FLAT8), jnp.float32),
        scratch_types=[
            pltpu.VMEM((_FLAT8,), jnp.float32),
            pltpu.VMEM((_NCH, _C), jnp.int32),
            pltpu.VMEM((_C * _NF,), jnp.int32),
            pltpu.VMEM((_C * _NF,), jnp.int32),
            pltpu.VMEM((_C, _H), jnp.float32),
            pltpu.VMEM((_C, _H), jnp.float32),
            pltpu.SemaphoreType.DMA,
            pltpu.SemaphoreType.DMA,
        ],
    )(fil128, src3, dx8)


def _touched_body(dst3, out, local, idx_d):
    w = _wid()
    pltpu.sync_copy(dst3.at[w], idx_d)

    def zslot(k, c2):
        local[pl.ds(k * 16, 16)] = jnp.zeros((16,), jnp.float32)
        return c2

    lax.fori_loop(0, _NPAD // 16, zslot, 0)
    ones = jnp.ones((16,), jnp.float32)

    def chunk(i, c2):
        for j in range(_C // 16):
            idx = idx_d[i, pl.ds(j * 16, 16)]
            plsc.store_scatter(local, [idx], ones)
        return c2

    lax.fori_loop(0, _NCH, chunk, 0)
    pltpu.sync_copy(local, out.at[w])


def _sc_touched(dsts3):
    return pl.kernel(
        _touched_body,
        mesh=_sc_mesh(),
        compiler_params=pltpu.CompilerParams(needs_layout_passes=False),
        out_type=jax.ShapeDtypeStruct((_NW, _NPAD), jnp.float32),
        scratch_types=[
            pltpu.VMEM((_NPAD,), jnp.float32),
            pltpu.VMEM((_NCH, _C), jnp.int32),
        ],
    )(dsts3)


def _fmax_body(*refs):
    out_ref = refs[-1]
    d = refs[0][0]
    for k in range(1, _NW):
        d = jnp.maximum(d, refs[k][0])
    out_ref[...] = d


def _flat_max32(a32r, rrows, brows):
    nb = rrows // brows
    return pl.pallas_call(
        _fmax_body,
        grid=(nb,),
        in_specs=[pl.BlockSpec((1, brows, _H), functools.partial(
            lambda k, i: (k, i, 0), k)) for k in range(_NW)],
        out_specs=pl.BlockSpec((brows, _H), lambda i: (i, 0)),
        out_shape=jax.ShapeDtypeStruct((rrows, _H), jnp.float32),
    )(*([a32r] * _NW))


def _death_merge_body(d_ref, fil128_ref, tch_ref, out_ref):
    fil = fil128_ref[...][:, :_NF]
    out_ref[...] = jnp.maximum(d_ref[...], fil * tch_ref[...])


def _death_merge(dmax, fil128, touched):
    nb = _N // _BN
    return pl.pallas_call(
        _death_merge_body,
        grid=(nb,),
        in_specs=[
            pl.BlockSpec((_BN, _NF), lambda i: (i, 0)),
            pl.BlockSpec((_BN, _H), lambda i: (i, 0)),
            pl.BlockSpec((_BN, 1), lambda i: (i, 0)),
        ],
        out_specs=pl.BlockSpec((_BN, _NF), lambda i: (i, 0)),
        out_shape=jax.ShapeDtypeStruct((_N, _NF), jnp.float32),
    )(dmax, fil128, touched)


def _dot(a, b, dims=None, prec=None):
    prec = prec or jax.lax.Precision.DEFAULT
    if dims is None:
        return jax.lax.dot(a, b, precision=prec,
                           preferred_element_type=jnp.float32)
    return jax.lax.dot_general(a, b, dims, precision=prec,
                               preferred_element_type=jnp.float32)


def _dotx(a, b, dims=None):
    return _dot(a, b, dims, prec=jax.lax.Precision.HIGHEST)


_BN2 = 2048


def _embed_body(ids_ref, tbl_ref, out_ref):
    onehot = (ids_ref[...] == jax.lax.broadcasted_iota(
        jnp.int32, (_BN2, _H), 1)).astype(jnp.float32)
    out_ref[...] = _dotx(onehot, tbl_ref[...])


def _embed_tc(ids2d, tbl_pad):
    nb = _NPAD // _BN2
    return pl.pallas_call(
        _embed_body,
        grid=(nb,),
        in_specs=[
            pl.BlockSpec((_BN2, 1), lambda i: (i, 0)),
            pl.BlockSpec((_H, _H), lambda i: (0, 0)),
        ],
        out_specs=pl.BlockSpec((_BN2, _H), lambda i: (i, 0)),
        out_shape=jax.ShapeDtypeStruct((_NPAD, _H), jnp.float32),
    )(ids2d, tbl_pad)


def _node_update_body(x0_ref, a0_ref, a1_ref, w0a_ref, w0b_ref,
                      wfil_ref, h_ref, fil_ref):
    x0 = x0_ref[...]
    agg = a0_ref[0] + a1_ref[0]
    t = jax.nn.relu(_dot(x0 + agg, w0a_ref[...]))
    h = jax.nn.relu(_dot(t, w0b_ref[...]))
    h_ref[...] = h
    fil = jax.nn.sigmoid(_dot(h, wfil_ref[...]))
    fil_ref[...] = jnp.tile(fil, (1, 16))


def _node_update(x0, agg2, w0a, w0b, wfil):
    nb = _NPAD // _BN2
    return pl.pallas_call(
        _node_update_body,
        grid=(nb,),
        in_specs=[
            pl.BlockSpec((_BN2, _H), lambda i: (i, 0)),
            pl.BlockSpec((1, _BN2, _H), lambda i: (0, i, 0)),
            pl.BlockSpec((1, _BN2, _H), lambda i: (1, i, 0)),
            pl.BlockSpec((_H, _H), lambda i: (0, 0)),
            pl.BlockSpec((_H, _H), lambda i: (0, 0)),
            pl.BlockSpec((_H, _NF), lambda i: (0, 0)),
        ],
        out_specs=[
            pl.BlockSpec((_BN2, _H), lambda i: (i, 0)),
            pl.BlockSpec((_BN2, _H), lambda i: (i, 0)),
        ],
        out_shape=[
            jax.ShapeDtypeStruct((_NPAD, _H), jnp.float32),
            jax.ShapeDtypeStruct((_NPAD, _H), jnp.float32),
        ],
    )(x0, agg2, agg2, w0a, w0b, wfil)


def _edge_mlp_body(x1_ref, bsum_ref, w1a_ref, w1b_ref, out_ref):
    z = _dot(x1_ref[...] + bsum_ref[...], w1a_ref[...])
    out_ref[...] = jax.nn.relu(_dot(jax.nn.relu(z), w1b_ref[...]))


def _edge_mlp(x1, bsum, w1a, w1b):
    nb = _EP // _BE
    return pl.pallas_call(
        _edge_mlp_body,
        grid=(nb,),
        in_specs=[
            pl.BlockSpec((_BE, _H), lambda i: (i, 0)),
            pl.BlockSpec((_BE, _H), lambda i: (i, 0)),
            pl.BlockSpec((_H, _H), lambda i: (0, 0)),
            pl.BlockSpec((_H, _H), lambda i: (0, 0)),
        ],
        out_specs=pl.BlockSpec((_BE, _H), lambda i: (i, 0)),
        out_shape=jax.ShapeDtypeStruct((_EP, _H), jnp.float32),
    )(x1, bsum, w1a, w1b)


def _edge_mlp0_body(bsum_ref, w1a_ref, w1b_ref, out_ref):
    z = _dot(bsum_ref[...] * 2.0, w1a_ref[...])
    out_ref[...] = jax.nn.relu(_dot(jax.nn.relu(z), w1b_ref[...]))


def _edge_mlp0(bsum, w1a, w1b):
    nb = _EP // _BE
    return pl.pallas_call(
        _edge_mlp0_body,
        grid=(nb,),
        in_specs=[
            pl.BlockSpec((_BE, _H), lambda i: (i, 0)),
            pl.BlockSpec((_H, _H), lambda i: (0, 0)),
            pl.BlockSpec((_H, _H), lambda i: (0, 0)),
        ],
        out_specs=pl.BlockSpec((_BE, _H), lambda i: (i, 0)),
        out_shape=jax.ShapeDtypeStruct((_EP, _H), jnp.float32),
    )(bsum, w1a, w1b)


def _edge_mlp_last_body(x1_ref, bsum_ref, w1a_ref, w1b_ref, bsrc_ref,
                        out_ref, g1_ref):
    z = _dot(x1_ref[...] + bsum_ref[...], w1a_ref[...])
    h = jax.nn.relu(_dot(jax.nn.relu(z), w1b_ref[...]))
    out_ref[...] = h
    onehot = (bsrc_ref[...] == jax.lax.broadcasted_iota(
        jnp.int32, (_BE, _G), 1)).astype(jnp.float32)

    @pl.when(pl.program_id(0) == 0)
    def _():
        g1_ref[...] = jnp.zeros_like(g1_ref)

    g1_ref[...] += _dotx(onehot, h, (((0,), (0,)), ((), ())))


def _edge_mlp_last(x1, bsum, w1a, w1b, bsrc2d):
    nb = _EP // _BE
    return pl.pallas_call(
        _edge_mlp_last_body,
        grid=(nb,),
        in_specs=[
            pl.BlockSpec((_BE, _H), lambda i: (i, 0)),
            pl.BlockSpec((_BE, _H), lambda i: (i, 0)),
            pl.BlockSpec((_H, _H), lambda i: (0, 0)),
            pl.BlockSpec((_H, _H), lambda i: (0, 0)),
            pl.BlockSpec((_BE, 1), lambda i: (i, 0)),
        ],
        out_specs=[
            pl.BlockSpec((_BE, _H), lambda i: (i, 0)),
            pl.BlockSpec((_G, _H), lambda i: (0, 0)),
        ],
        out_shape=[
            jax.ShapeDtypeStruct((_EP, _H), jnp.float32),
            jax.ShapeDtypeStruct((_G, _H), jnp.float32),
        ],
    )(x1, bsum, w1a, w1b, bsrc2d)


def _pe_common(fil128_ref, death_ref, r_ref, w0r_ref, w1r_ref, wp2e_ref,
               batch_ref):
    fil = fil128_ref[...][:, :_NF]
    death = death_ref[...]
    fil128 = _dot(fil, r_ref[...])
    death128 = _dot(death, r_ref[...])
    m = jax.nn.relu(fil128 * w0r_ref[...] + death128 * w1r_ref[...])
    pe = _dot(m, wp2e_ref[...])
    onehot = (batch_ref[...] == jax.lax.broadcasted_iota(
        jnp.int32, (_BN, _G), 1)).astype(jnp.float32)
    return pe, onehot


def _pe_body(fil16_ref, death_ref, r_ref, w0r_ref, w1r_ref, wp2e_ref,
             batch_ref, ph_ref):
    pe, onehot = _pe_common(fil16_ref, death_ref, r_ref, w0r_ref,
                            w1r_ref, wp2e_ref, batch_ref)

    @pl.when(pl.program_id(0) == 0)
    def _():
        ph_ref[...] = jnp.zeros_like(ph_ref)

    ph_ref[...] += _dotx(onehot, pe, (((0,), (0,)), ((), ())))


def _pe_last_body(fil16_ref, death_ref, r_ref, w0r_ref, w1r_ref,
                  wp2e_ref, batch_ref, x0_ref, ph_ref, g0_ref):
    pe, onehot = _pe_common(fil16_ref, death_ref, r_ref, w0r_ref,
                            w1r_ref, wp2e_ref, batch_ref)

    @pl.when(pl.program_id(0) == 0)
    def _():
        ph_ref[...] = jnp.zeros_like(ph_ref)
        g0_ref[...] = jnp.zeros_like(g0_ref)

    ph_ref[...] += _dotx(onehot, pe, (((0,), (0,)), ((), ())))
    g0_ref[...] += _dotx(onehot, x0_ref[...], (((0,), (0,)), ((), ())))


_pe_common_specs = [
    pl.BlockSpec((_BN, _H), lambda i: (i, 0)),
    pl.BlockSpec((_BN, _NF), lambda i: (i, 0)),
    pl.BlockSpec((_NF, _H), lambda i: (0, 0)),
    pl.BlockSpec((1, _H), lambda i: (0, 0)),
    pl.BlockSpec((1, _H), lambda i: (0, 0)),
    pl.BlockSpec((_H, _OPH), lambda i: (0, 0)),
    pl.BlockSpec((_BN, 1), lambda i: (i, 0)),
]


def _pe_readout(fil16, death, r, w0r, w1r, wp2e, batch2d):
    nb = _N // _BN
    return pl.pallas_call(
        _pe_body,
        grid=(nb,),
        in_specs=list(_pe_common_specs),
        out_specs=pl.BlockSpec((_G, _OPH), lambda i: (0, 0)),
        out_shape=jax.ShapeDtypeStruct((_G, _OPH), jnp.float32),
    )(fil16, death, r, w0r, w1r, wp2e, batch2d)


def _pe_readout_last(fil16, death, r, w0r, w1r, wp2e, batch2d, x0):
    nb = _N // _BN
    return pl.pallas_call(
        _pe_last_body,
        grid=(nb,),
        in_specs=list(_pe_common_specs) + [
            pl.BlockSpec((_BN, _H), lambda i: (i, 0)),
        ],
        out_specs=[
            pl.BlockSpec((_G, _OPH), lambda i: (0, 0)),
            pl.BlockSpec((_G, _H), lambda i: (0, 0)),
        ],
        out_shape=[
            jax.ShapeDtypeStruct((_G, _OPH), jnp.float32),
            jax.ShapeDtypeStruct((_G, _H), jnp.float32),
        ],
    )(fil16, death, r, w0r, w1r, wp2e, batch2d, x0)


def _head_body(g0_ref, g1_ref, ph0_ref, ph1_ref, ph2_ref, wl0_ref, b0_ref,
               wl1_ref, b1_ref, w2a_ref, w2b_ref, b2_ref, out_ref):
    z = (jax.nn.relu(_dot(g0_ref[...], wl0_ref[...]) + b0_ref[...]) +
         jax.nn.relu(_dot(g1_ref[...], wl1_ref[...]) + b1_ref[...]))
    ph = (ph0_ref[...] + ph1_ref[...] + ph2_ref[...]) * (1.0 / 3.0)
    out_ref[...] = _dot(z, w2a_ref[...]) + _dot(ph, w2b_ref[...]) + b2_ref[...]


def _head(g0, g1, ph0, ph1, ph2, wl0, b0, wl1, b1, w2a, w2b, b2):
    return pl.pallas_call(
        _head_body,
        out_shape=jax.ShapeDtypeStruct((_G, 1), jnp.float32),
    )(g0, g1, ph0, ph1, ph2, wl0, b0, wl1, b1, w2a, w2b, b2)


def kernel(atom_ids, edge_index, batch, atom_table, W0a, W0b, W1a, W1b, Wfil,
           Wp1, Wp2, Wlin1_0, b1_0, Wlin1_1, b1_1, Wlin2, blin2):
    src = edge_index[0]
    dst = edge_index[1]

    pad = _EP - _E
    spad = jnp.full((pad,), _N, jnp.int32)
    srcs = jnp.concatenate([src.astype(jnp.int32), spad])
    dsts = jnp.concatenate([dst.astype(jnp.int32), spad])
    srcs3 = srcs.reshape(_NW, _NCH, _C)
    dsts3 = dsts.reshape(_NW, _NCH, _C)
    srcs3c = srcs.reshape(_NW, _NCH2, _C2)
    dsts3c = dsts.reshape(_NW, _NCH2, _C2)
    dx8 = ((dsts[:, None] * _NF) +
           jnp.arange(_NF, dtype=jnp.int32)[None, :]).reshape(
               _NW, _NCH, _C * _NF)

    npad = _NPAD - _N
    ids2d = jnp.concatenate([atom_ids.astype(jnp.int32),
                             jnp.zeros((npad,), jnp.int32)]).reshape(_NPAD, 1)
    tbl_pad = jnp.zeros((_H, _H), jnp.float32).at[:119].set(atom_table)
    x0 = _embed_tc(ids2d, tbl_pad)
    batchp = jnp.concatenate([batch.astype(jnp.int32),
                              jnp.full((npad,), _G, jnp.int32)])
    batch2d = batchp.reshape(_NPAD, 1)
    bsrc2d = _sc_bsrc(batchp, srcs3).reshape(_EP, 1)

    eye8 = jnp.eye(_NF, dtype=jnp.float32)
    r = jnp.repeat(eye8, _FH, axis=1)
    w0r = jnp.tile(Wp1[:, 0, :], (1, _NF)).reshape(_L, 1, _NF * _FH)
    w1r = jnp.tile(Wp1[:, 1, :], (1, _NF)).reshape(_L, 1, _NF * _FH)
    wp2e = jnp.tile(Wp2, (1, _NF, 1)) / float(_NF)

    t32 = _sc_touched(dsts3).reshape(_NW, _NPAD // _H, _H)
    touched = _flat_max32(t32, _NPAD // _H, _NPAD // _H).reshape(_NPAD, 1)

    ph_list = []
    g0 = g1 = None
    x1 = None
    for l in range(_L):
        agg2, bsum = _sc_agg_bsum(x0, srcs3c, dsts3c)
        h, fil128 = _node_update(x0, agg2, W0a[l], W0b[l], Wfil[l])
        if l == 0:
            x1 = _edge_mlp0(bsum, W1a[l], W1b[l])
        elif l == _L - 1:
            x1, g1 = _edge_mlp_last(x1, bsum, W1a[l], W1b[l], bsrc2d)
        else:
            x1 = _edge_mlp(x1, bsum, W1a[l], W1b[l])
        x0 = h
        death32 = _sc_death(fil128, srcs3, dx8).reshape(
            _NW, _FLAT8 // _H, _H)
        dmax = _flat_max32(death32, _FLAT8 // _H, _H).reshape(_NPAD, _NF)
        death = _death_merge(dmax, fil128, touched)
        if l == _L - 1:
            ph_l, g0 = _pe_readout_last(fil128, death, r, w0r[l], w1r[l],
                                        wp2e[l], batch2d, x0)
        else:
            ph_l = _pe_readout(fil128, death, r, w0r[l], w1r[l], wp2e[l],
                               batch2d)
        ph_list.append(ph_l)

    b0_2d = b1_0.reshape(1, 2 * _H)
    b1_2d = b1_1.reshape(1, 2 * _H)
    w2a = Wlin2[:2 * _H]
    w2b = Wlin2[2 * _H:]
    b2_2d = blin2.reshape(1, 1)
    return _head(g0, g1, ph_list[0], ph_list[1], ph_list[2],
                 Wlin1_0, b0_2d, Wlin1_1, b1_2d, w2a, w2b, b2_2d)
